# Initial kernel scaffold; baseline (speedup 1.0000x reference)
import functools

import jax
import jax.numpy as jnp
from jax import lax
from jax.experimental import pallas as pl
from jax.experimental.pallas import tpu as pltpu
from jax.experimental.pallas import tpu_sc as plsc

N, E, F, H, B, C = 10000, 320000, 128, 256, 64, 10

NC, NS, L = 2, 16, 16
NPAD = 10240
CH = 80
SKEY_PAD = 1024
SG = 544
INT_MIN = jnp.int32(-2147483648)

_mesh = functools.partial(
    plsc.VectorSubcoreMesh, core_axis_name="c", subcore_axis_name="s",
    num_cores=NC, num_subcores=NS)


def _f32key_to_i32(b):
  return jnp.where(b >= 0, b, INT_MIN - b)


def _sc_degree(dst):
  n_chunk = E // (NC * NS * CH)

  def body(dst_hbm, out_hbm, ones_v, dst_v, acc, wb):
    c = lax.axis_index("c")
    s = lax.axis_index("s")

    def fill(i, _):
      ones_v[i, :] = jnp.ones((L,), jnp.float32)
      return 0
    lax.fori_loop(0, CH, fill, 0)

    rows0 = s * (NPAD // NS)

    def zrow(i, _):
      wb[i, 0:L] = jnp.zeros((L,), jnp.float32)
      return 0
    lax.fori_loop(0, 128, zrow, 0)
    for t in range(5):
      pltpu.sync_copy(wb, acc.at[pl.ds(rows0 + 128 * t, 128)])
    plsc.subcore_barrier()

    wid = s * NC + c
    base0 = wid * (E // (NC * NS))

    def chunk(j, _):
      pltpu.sync_copy(dst_hbm.at[pl.ds(base0 + j * CH, CH)], dst_v)
      pltpu.sync_copy(ones_v, acc.at[dst_v], add=True)
      return 0
    lax.fori_loop(0, n_chunk, chunk, 0)
    plsc.subcore_barrier()

    for t in range(5):
      pltpu.sync_copy(acc.at[pl.ds(rows0 + 128 * t, 128)], wb)
      pltpu.sync_copy(wb, out_hbm.at[pl.ds(c * NPAD + rows0 + 128 * t, 128)])

  k = pl.kernel(
      body,
      out_type=jax.ShapeDtypeStruct((NC * NPAD, L), jnp.float32),
      mesh=_mesh(),
      scratch_types=[
          pltpu.VMEM((CH, L), jnp.float32),
          pltpu.VMEM((CH,), jnp.int32),
          pltpu.VMEM_SHARED((NPAD, L), jnp.float32),
          pltpu.VMEM((128, L), jnp.float32),
      ],
  )
  return k(dst)


def _sc_segsum16(tab16, src, dst):
  n_chunk = E // (NC * NS * CH)

  def body(tab_hbm, src_hbm, dst_hbm, out_hbm, src_v, dst_v, rows_v, acc, wb,
           sem):
    c = lax.axis_index("c")
    s = lax.axis_index("s")
    rows0 = s * (NPAD // NS)

    def zrow(i, _):
      wb[i, 0:L] = jnp.zeros((L,), jnp.float32)
      return 0
    lax.fori_loop(0, 128, zrow, 0)
    for t in range(5):
      pltpu.sync_copy(wb, acc.at[pl.ds(rows0 + 128 * t, 128)])
    plsc.subcore_barrier()

    wid = s * NC + c
    base0 = wid * (E // (NC * NS))

    def chunk(j, _):
      base = base0 + j * CH
      pltpu.sync_copy(src_hbm.at[pl.ds(base, CH)], src_v)
      pltpu.sync_copy(dst_hbm.at[pl.ds(base, CH)], dst_v)
      pltpu.async_copy(tab_hbm.at[src_v], rows_v, sem).wait()
      pltpu.sync_copy(rows_v, acc.at[dst_v], add=True)
      return 0
    lax.fori_loop(0, n_chunk, chunk, 0)
    plsc.subcore_barrier()

    for t in range(5):
      pltpu.sync_copy(acc.at[pl.ds(rows0 + 128 * t, 128)], wb)
      pltpu.sync_copy(wb, out_hbm.at[pl.ds(c * NPAD + rows0 + 128 * t, 128)])

  k = pl.kernel(
      body,
      out_type=jax.ShapeDtypeStruct((NC * NPAD, L), jnp.float32),
      mesh=_mesh(),
      scratch_types=[
          pltpu.VMEM((CH,), jnp.int32),
          pltpu.VMEM((CH,), jnp.int32),
          pltpu.VMEM((CH, L), jnp.float32),
          pltpu.VMEM_SHARED((NPAD, L), jnp.float32),
          pltpu.VMEM((128, L), jnp.float32),
          pltpu.SemaphoreType.DMA,
      ],
  )
  return k(tab16, src, dst)


def _sc_segsum_feat(tab2, src, dst, dh):
  n_pair = E // (NS * CH * 2)

  def body(tab_hbm, src_hbm, dst_hbm, out_hbm,
           src0, dst0, rows0, src1, dst1, rows1, acc, wb, sem0, sem1):
    c = lax.axis_index("c")
    s = lax.axis_index("s")
    rows_base = s * (NPAD // NS)

    nz = dh // L

    def zrow(i, _):
      def zlane(k, __):
        wb[i, pl.ds(k * L, L)] = jnp.zeros((L,), jnp.float32)
        return 0
      lax.fori_loop(0, nz, zlane, 0)
      return 0
    lax.fori_loop(0, 128, zrow, 0)
    for t in range(5):
      pltpu.sync_copy(wb, acc.at[pl.ds(rows_base + 128 * t, 128)])
    plsc.subcore_barrier()

    base0 = s * (E // NS)
    coff = c * N

    def adjust(sv):
      for kk in range(CH // L):
        sl = pl.ds(kk * L, L)
        sv[sl] = sv[sl] + jnp.full((L,), coff, jnp.int32)

    def chunk(j, _):
      ba = base0 + (2 * j) * CH
      bb = base0 + (2 * j + 1) * CH
      pltpu.sync_copy(src_hbm.at[pl.ds(ba, CH)], src0)
      pltpu.sync_copy(dst_hbm.at[pl.ds(ba, CH)], dst0)
      adjust(src0)
      cp0 = pltpu.async_copy(tab_hbm.at[src0], rows0, sem0)
      pltpu.sync_copy(src_hbm.at[pl.ds(bb, CH)], src1)
      pltpu.sync_copy(dst_hbm.at[pl.ds(bb, CH)], dst1)
      adjust(src1)
      cp1 = pltpu.async_copy(tab_hbm.at[src1], rows1, sem1)
      cp0.wait()
      pltpu.sync_copy(rows0, acc.at[dst0], add=True)
      cp1.wait()
      pltpu.sync_copy(rows1, acc.at[dst1], add=True)
      return 0
    lax.fori_loop(0, n_pair, chunk, 0)
    plsc.subcore_barrier()

    for t in range(5):
      pltpu.sync_copy(acc.at[pl.ds(rows_base + 128 * t, 128)], wb)
      pltpu.sync_copy(
          wb, out_hbm.at[pl.ds(c * NPAD + rows_base + 128 * t, 128)])

  k = pl.kernel(
      body,
      out_type=jax.ShapeDtypeStruct((NC * NPAD, dh), jnp.float32),
      mesh=_mesh(),
      scratch_types=[
          pltpu.VMEM((CH,), jnp.int32),
          pltpu.VMEM((CH,), jnp.int32),
          pltpu.VMEM((CH, dh), jnp.float32),
          pltpu.VMEM((CH,), jnp.int32),
          pltpu.VMEM((CH,), jnp.int32),
          pltpu.VMEM((CH, dh), jnp.float32),
          pltpu.VMEM_SHARED((NPAD, dh), jnp.float32),
          pltpu.VMEM((128, dh), jnp.float32),
          pltpu.SemaphoreType.DMA,
          pltpu.SemaphoreType.DMA,
      ],
  )
  return k(tab2, src, dst)


def _sc_topk(skey_pad, sc128):
  nv = SG // L

  def body(skey_hbm, sc_hbm, tlo_hbm, thi_hbm, u_hbm,
           kbuf, keyi, obuf, scm):
    c = lax.axis_index("c")
    s = lax.axis_index("s")
    wid = s * NC + c
    pltpu.sync_copy(sc_hbm, scm)

    iota = lax.iota(jnp.int32, L)

    for g_loc in range(2):
      g = wid * 2 + g_loc
      start = scm[g]
      count = scm[B + g]
      a = lax.bitwise_and(start, jnp.int32(-8))
      off = start - a
      pltpu.sync_copy(skey_hbm.at[pl.ds(a, SG)], kbuf)

      def conv(j, _):
        sl = pl.ds(j * L, L)
        bits = plsc.bitcast(kbuf[sl], jnp.int32)
        keyi[sl] = _f32key_to_i32(bits)
        return 0
      lax.fori_loop(0, nv, conv, 0)

      offv = jnp.full((L,), off, jnp.int32)
      cntv = jnp.full((L,), count, jnp.int32)
      kq = 4 * count + 4
      kper = lax.shift_right_logical(kq * 52429, 18)
      kperv = jnp.full((L,), kper, jnp.int32)

      def count_pred(pred):
        def inner(j, acc):
          sl = pl.ds(j * L, L)
          kv = keyi[sl]
          pos = jnp.full((L,), j * L, jnp.int32) + iota - offv
          valid = (pos >= 0) & (pos < cntv)
          return acc + plsc.all_reduce_population_count(pred(kv, pos) & valid)
        return lax.fori_loop(0, nv, inner, jnp.zeros((L,), jnp.int32))

      def bis_a(it, lh):
        lo, hi = lh
        mid = (lax.shift_right_arithmetic(lo, 1)
               + lax.shift_right_arithmetic(hi, 1)
               + (lo & hi & 1))
        ge = count_pred(lambda kv, pos: kv <= mid) >= kperv
        return (jnp.where(ge, lo, mid + 1), jnp.where(ge, mid, hi))

      lo0 = jnp.full((L,), INT_MIN, jnp.int32)
      hi0 = jnp.full((L,), jnp.int32(2147483647), jnp.int32)
      lo, hi = lax.fori_loop(0, 32, bis_a, (lo0, hi0))
      t = lo

      strict = count_pred(lambda kv, pos: kv < t)
      r = kperv - strict

      def bis_b(it, lh):
        lo2, hi2 = lh
        mid = lax.shift_right_arithmetic(lo2 + hi2, 1)
        ge = count_pred(lambda kv, pos: (kv == t) & (pos < mid)) >= r
        return (jnp.where(ge, lo2, mid + 1), jnp.where(ge, mid, hi2))

      lo2, hi2 = lax.fori_loop(
          0, 12, bis_b,
          (jnp.zeros((L,), jnp.int32), cntv))
      u = jnp.full((L,), start, jnp.int32) + lo2

      tlo = (t & jnp.full((L,), 65535, jnp.int32)).astype(jnp.float32)
      thi = lax.shift_right_arithmetic(t, 16).astype(jnp.float32)
      obuf[0:L] = tlo
      pltpu.sync_copy(obuf, tlo_hbm.at[g])
      obuf[0:L] = thi
      pltpu.sync_copy(obuf, thi_hbm.at[g])
      obuf[0:L] = u.astype(jnp.float32)
      pltpu.sync_copy(obuf, u_hbm.at[g])

  k = pl.kernel(
      body,
      out_type=(
          jax.ShapeDtypeStruct((B, L), jnp.float32),
          jax.ShapeDtypeStruct((B, L), jnp.float32),
          jax.ShapeDtypeStruct((B, L), jnp.float32),
      ),
      mesh=_mesh(),
      scratch_types=[
          pltpu.VMEM((SG,), jnp.float32),
          pltpu.VMEM((SG,), jnp.int32),
          pltpu.VMEM((L,), jnp.float32),
          pltpu.SMEM((2 * B,), jnp.int32),
      ],
  )
  return k(skey_pad, sc128)


RB = 1000


def _conv_body(weighted, dh, agg_ref, deg_ref, w_ref, xin_ref, wrel_ref,
               b_ref, wroot_ref, batch_ref, h_ref, xsum_ref):
  i = pl.program_id(0)
  d = deg_ref[0, :, 0:1] + deg_ref[1, :, 0:1]
  alo = agg_ref[0]
  ahi = agg_ref[1]
  if weighted:
    w = w_ref[:, 0:1]
    den = jnp.maximum(d * w, 1.0)
    alo = (alo * w) / den
    ahi = (ahi * w) / den
  else:
    den = jnp.maximum(d, 1.0)
    alo = alo / den
    ahi = ahi / den
  acc = jnp.dot(alo, wrel_ref[:dh], preferred_element_type=jnp.float32)
  acc = acc + jnp.dot(ahi, wrel_ref[dh:], preferred_element_type=jnp.float32)
  acc = acc + b_ref[0:1, :]
  acc = acc + jnp.dot(xin_ref[0], wroot_ref[:dh],
                      preferred_element_type=jnp.float32)
  acc = acc + jnp.dot(xin_ref[1], wroot_ref[dh:],
                      preferred_element_type=jnp.float32)
  h = jnp.maximum(acc, 0.0)
  if weighted:
    h = h * w
  h_ref[0] = h[:, :128]
  h_ref[1] = h[:, 128:]
  gcol = lax.broadcasted_iota(jnp.int32, (RB, B), 1)
  onehot = (batch_ref[:, 0:1] == gcol).astype(jnp.float32)
  part = lax.dot_general(onehot, h, (((0,), (0,)), ((), ())),
                         preferred_element_type=jnp.float32)

  @pl.when(i == 0)
  def _():
    xsum_ref[...] = jnp.zeros_like(xsum_ref)

  xsum_ref[...] += part


def _tc_conv(agg2, degsm, w16, xin2, wrel, b, wroot, batch2, weighted, dh):
  grid = (N // RB,)
  in_specs = [
      pl.BlockSpec((2, RB, dh), lambda i: (0, i, 0)),
      pl.BlockSpec((2, RB, L), lambda i: (0, i, 0)),
      pl.BlockSpec((RB, L), lambda i: (i, 0)),
      pl.BlockSpec((2, RB, dh), lambda i: (0, i, 0)),
      pl.BlockSpec((2 * dh, H), lambda i: (0, 0)),
      pl.BlockSpec((1, H), lambda i: (0, 0)),
      pl.BlockSpec((2 * dh, H), lambda i: (0, 0)),
      pl.BlockSpec((RB, 1), lambda i: (i, 0)),
  ]
  out_specs = [
      pl.BlockSpec((2, RB, 128), lambda i: (0, i, 0)),
      pl.BlockSpec((B, H), lambda i: (0, 0)),
  ]
  out_shape = [
      jax.ShapeDtypeStruct((2, N, 128), jnp.float32),
      jax.ShapeDtypeStruct((B, H), jnp.float32),
  ]
  return pl.pallas_call(
      functools.partial(_conv_body, weighted, dh),
      grid=grid, in_specs=in_specs, out_specs=out_specs, out_shape=out_shape,
  )(agg2, degsm, w16, xin2, wrel, b, wroot, batch2)


def _score_body(h_ref, p_ref, batch_ref, bfull_ref, score_ref, skey_ref,
                sc_ref):
  i = pl.program_id(0)
  p = p_ref[...]
  nrm = jnp.sqrt(jnp.sum(p * p))
  hp = jnp.dot(h_ref[0], p[:128, :], preferred_element_type=jnp.float32)
  hp = hp + jnp.dot(h_ref[1], p[128:, :], preferred_element_type=jnp.float32)
  s = jnp.tanh(hp / nrm)
  score_ref[...] = s
  skey_ref[...] = batch_ref[...].astype(jnp.float32) * 4.0 - s

  @pl.when(i == 0)
  def _():
    gcol = lax.broadcasted_iota(jnp.int32, (N, B), 1)
    onehot = (bfull_ref[:, 0:1] == gcol).astype(jnp.float32)
    counts = jnp.sum(onehot, axis=0, keepdims=True)
    rr = lax.broadcasted_iota(jnp.int32, (B, B), 0)
    cc = lax.broadcasted_iota(jnp.int32, (B, B), 1)
    tri = (rr < cc).astype(jnp.float32)
    starts = jnp.dot(counts, tri, preferred_element_type=jnp.float32)
    sc_ref[...] = jnp.concatenate([starts, counts], axis=1).astype(jnp.int32)


def _tc_score(h2, p0, batch2):
  grid = (N // RB,)
  return pl.pallas_call(
      _score_body,
      grid=grid,
      in_specs=[
          pl.BlockSpec((2, RB, 128), lambda i: (0, i, 0)),
          pl.BlockSpec((H, 1), lambda i: (0, 0)),
          pl.BlockSpec((RB, 1), lambda i: (i, 0)),
          pl.BlockSpec((N, 1), lambda i: (0, 0)),
      ],
      out_specs=[
          pl.BlockSpec((RB, 1), lambda i: (i, 0)),
          pl.BlockSpec((RB, 1), lambda i: (i, 0)),
          pl.BlockSpec((1, 2 * B), lambda i: (0, 0)),
      ],
      out_shape=[
          jax.ShapeDtypeStruct((N, 1), jnp.float32),
          jax.ShapeDtypeStruct((N, 1), jnp.float32),
          jax.ShapeDtypeStruct((1, 2 * B), jnp.int32),
      ],
  )(h2, p0, batch2, batch2)


def _mask_body(h_ref, score_ref, skey_ref, batch_ref, th_ref, h3m_ref,
               m16_ref):
  i = pl.program_id(0)
  gcol = lax.broadcasted_iota(jnp.int32, (RB, B), 1)
  onehot = (batch_ref[:, 0:1] == gcol).astype(jnp.float32)
  g3 = jnp.dot(onehot, th_ref[...], preferred_element_type=jnp.float32)
  tl = g3[:, 0:1].astype(jnp.int32)
  th = g3[:, 1:2].astype(jnp.int32)
  uu = g3[:, 2:3]
  t_node = th * 65536 + tl
  bits = lax.bitcast_convert_type(skey_ref[...], jnp.int32)
  key = _f32key_to_i32(bits)
  idxrow = (lax.broadcasted_iota(jnp.int32, (RB, 1), 0)
            + i * RB).astype(jnp.float32)
  m = ((key < t_node) | ((key == t_node) & (idxrow < uu))).astype(jnp.float32)
  hm = score_ref[...] * m
  h3m_ref[0] = h_ref[0] * hm
  h3m_ref[1] = h_ref[1] * hm
  m16_ref[...] = jnp.broadcast_to(m, (RB, L))


def _tc_mask(h2, score, skey, batch2, th3):
  grid = (N // RB,)
  return pl.pallas_call(
      _mask_body,
      grid=grid,
      in_specs=[
          pl.BlockSpec((2, RB, 128), lambda i: (0, i, 0)),
          pl.BlockSpec((RB, 1), lambda i: (i, 0)),
          pl.BlockSpec((RB, 1), lambda i: (i, 0)),
          pl.BlockSpec((RB, 1), lambda i: (i, 0)),
          pl.BlockSpec((B, 3), lambda i: (0, 0)),
      ],
      out_specs=[
          pl.BlockSpec((2, RB, 128), lambda i: (0, i, 0)),
          pl.BlockSpec((RB, L), lambda i: (i, 0)),
      ],
      out_shape=[
          jax.ShapeDtypeStruct((2, N, 128), jnp.float32),
          jax.ShapeDtypeStruct((N, L), jnp.float32),
      ],
  )(h2, score, skey, batch2, th3)


def _mlp_body(x0, x1, x2, x3, w1, b1, w2, b2, w3, b3, out_ref):
  z = jnp.dot(x0[...], w1[:H], preferred_element_type=jnp.float32)
  z = z + jnp.dot(x1[...], w1[H:2 * H], preferred_element_type=jnp.float32)
  z = z + jnp.dot(x2[...], w1[2 * H:3 * H], preferred_element_type=jnp.float32)
  z = z + jnp.dot(x3[...], w1[3 * H:], preferred_element_type=jnp.float32)
  z = jnp.maximum(z + b1[0:1, :], 0.0)
  z = jnp.maximum(jnp.dot(z, w2[...], preferred_element_type=jnp.float32)
                  + b2[0:1, :], 0.0)
  z = jnp.dot(z, w3[...], preferred_element_type=jnp.float32) + b3[0:1, :]
  mx = jnp.max(z, axis=1, keepdims=True)
  sh = z - mx
  out_ref[...] = sh - jnp.log(jnp.sum(jnp.exp(sh), axis=1, keepdims=True))


def _tc_mlp(xs, params):
  return pl.pallas_call(
      _mlp_body,
      out_shape=jax.ShapeDtypeStruct((B, C), jnp.float32),
  )(xs[0], xs[1], xs[2], xs[3],
    params['W1'], params['b1'].reshape(1, H),
    params['W2'], params['b2'].reshape(1, H // 2),
    params['W3'], params['b3'].reshape(1, C))


def kernel(x, params, edge_index, batch):
  src = edge_index[0]
  dst = edge_index[1]
  batch2 = batch.reshape(N, 1)

  x_st = jnp.stack([x[:, :F // 2], x[:, F // 2:]])
  x_flat = x_st.reshape(2 * N, F // 2)

  deg = _sc_degree(dst).reshape(2, NPAD, L)

  agg1 = _sc_segsum_feat(x_flat, src, dst, F // 2).reshape(2, NPAD, F // 2)
  h1, xs0 = _tc_conv(agg1, deg, deg[0, :N], x_st, params['W_rel1'],
                     params['b_rel1'].reshape(1, H), params['W_root1'],
                     batch2, False, F // 2)

  h1_flat = h1.reshape(2 * N, 128)
  agg2 = _sc_segsum_feat(h1_flat, src, dst, 128).reshape(2, NPAD, 128)
  h2, xs1 = _tc_conv(agg2, deg, deg[0, :N], h1, params['W_rel2'],
                     params['b_rel2'].reshape(1, H), params['W_root2'],
                     batch2, False, 128)

  score, skey, sc128 = _tc_score(h2, params['p0'].reshape(H, 1), batch2)
  skey_pad = jnp.concatenate(
      [skey.reshape(N), jnp.full((SKEY_PAD,), 1e30, jnp.float32)])
  tlo16, thi16, u16 = _sc_topk(skey_pad, sc128.reshape(2 * B))
  th3 = jnp.concatenate(
      [tlo16[:, 0:1], thi16[:, 0:1], u16[:, 0:1]], axis=1)

  h3m, m16 = _tc_mask(h2, score, skey, batch2, th3)

  sm = _sc_segsum16(m16, src, dst).reshape(2, NPAD, L)

  h3m_flat = h3m.reshape(2 * N, 128)
  agg3 = _sc_segsum_feat(h3m_flat, src, dst, 128).reshape(2, NPAD, 128)
  h4m, xs2 = _tc_conv(agg3, sm, m16, h3m, params['W_rel3'],
                      params['b_rel3'].reshape(1, H), params['W_root3'],
                      batch2, True, 128)

  h4m_flat = h4m.reshape(2 * N, 128)
  agg4 = _sc_segsum_feat(h4m_flat, src, dst, 128).reshape(2, NPAD, 128)
  _, xs3 = _tc_conv(agg4, sm, m16, h4m, params['W_rel4'],
                    params['b_rel4'].reshape(1, H), params['W_root4'],
                    batch2, True, 128)

  return _tc_mlp([xs0, xs1, xs2, xs3], params)


# speedup vs baseline: 6.2836x; 6.2836x over previous
import functools

import jax
import jax.numpy as jnp
from jax import lax
from jax.experimental import pallas as pl
from jax.experimental.pallas import tpu as pltpu
from jax.experimental.pallas import tpu_sc as plsc

N, E, F, H, B, C = 10000, 320000, 128, 256, 64, 10

NC, NS, L = 2, 16, 16
NPAD = 10240
CH = 80
SKEY_PAD = 1024
SG = 544
INT_MIN = -2147483648

_mesh = functools.partial(
    plsc.VectorSubcoreMesh, core_axis_name="c", subcore_axis_name="s",
    num_cores=NC, num_subcores=NS)


def _f32key_to_i32(b):
  return jnp.where(b >= 0, b, INT_MIN - b)


def _sc_degree(ones128, zrs128, dst):
  n_chunk = E // (NC * NS * CH)
  dh = 128

  def body(ones_hbm, zrs_hbm, dst_hbm, out_hbm, ones_v, dst_v, acc, wb):
    c = lax.axis_index("c")
    s = lax.axis_index("s")
    rows0 = s * (NPAD // NS)

    pltpu.sync_copy(ones_hbm, ones_v)
    pltpu.sync_copy(zrs_hbm, wb)
    for t in range(5):
      pltpu.sync_copy(wb, acc.at[pl.ds(rows0 + 128 * t, 128)])
    plsc.subcore_barrier()

    wid = s * NC + c
    base0 = wid * (E // (NC * NS))

    def chunk(j, _):
      pltpu.sync_copy(dst_hbm.at[pl.ds(base0 + j * CH, CH)], dst_v)
      pltpu.sync_copy(ones_v, acc.at[dst_v], add=True)
      return 0
    lax.fori_loop(0, n_chunk, chunk, 0)
    plsc.subcore_barrier()

    for t in range(5):
      pltpu.sync_copy(acc.at[pl.ds(rows0 + 128 * t, 128)], wb)
      pltpu.sync_copy(wb, out_hbm.at[pl.ds(c * NPAD + rows0 + 128 * t, 128)])

  k = pl.kernel(
      body,
      out_type=jax.ShapeDtypeStruct((NC * NPAD, dh), jnp.float32),
      mesh=_mesh(),
      scratch_types=[
          pltpu.VMEM((CH, dh), jnp.float32),
          pltpu.VMEM((CH,), jnp.int32),
          pltpu.VMEM_SHARED((NPAD, dh), jnp.float32),
          pltpu.VMEM((128, dh), jnp.float32),
      ],
  )
  return k(ones128, zrs128, dst)


def _sc_segsum_feat(tab, src, dst, feat_split):
  dh = 128
  per_worker = E // NS if feat_split else E // (NC * NS)
  nch = per_worker // CH
  n_pair = nch // 2
  has_tail = (nch % 2) == 1

  def body(tab_hbm, src_hbm, dst_hbm, out_hbm,
           src0, dst0, rows0, src1, dst1, rows1, acc, wb, sem0, sem1):
    c = lax.axis_index("c")
    s = lax.axis_index("s")
    rows_base = s * (NPAD // NS)

    nz = dh // L

    def zrow(i, _):
      def zlane(k, __):
        wb[i, pl.ds(k * L, L)] = jnp.zeros((L,), jnp.float32)
        return 0
      lax.fori_loop(0, nz, zlane, 0)
      return 0
    lax.fori_loop(0, 128, zrow, 0)
    for t in range(5):
      pltpu.sync_copy(wb, acc.at[pl.ds(rows_base + 128 * t, 128)])
    plsc.subcore_barrier()

    if feat_split:
      base0 = s * per_worker
      coff = c * N
    else:
      base0 = (s * NC + c) * per_worker
      coff = None

    def adjust(sv):
      if coff is None:
        return
      for kk in range(CH // L):
        sl = pl.ds(kk * L, L)
        sv[sl] = sv[sl] + jnp.full((L,), coff, jnp.int32)

    def do_pair(ba, bb):
      pltpu.sync_copy(src_hbm.at[pl.ds(ba, CH)], src0)
      pltpu.sync_copy(dst_hbm.at[pl.ds(ba, CH)], dst0)
      adjust(src0)
      cp0 = pltpu.async_copy(tab_hbm.at[src0], rows0, sem0)
      pltpu.sync_copy(src_hbm.at[pl.ds(bb, CH)], src1)
      pltpu.sync_copy(dst_hbm.at[pl.ds(bb, CH)], dst1)
      adjust(src1)
      cp1 = pltpu.async_copy(tab_hbm.at[src1], rows1, sem1)
      cp0.wait()
      pltpu.sync_copy(rows0, acc.at[dst0], add=True)
      cp1.wait()
      pltpu.sync_copy(rows1, acc.at[dst1], add=True)

    def chunk(j, _):
      do_pair(base0 + (2 * j) * CH, base0 + (2 * j + 1) * CH)
      return 0
    lax.fori_loop(0, n_pair, chunk, 0)
    if has_tail:
      ba = base0 + (nch - 1) * CH
      pltpu.sync_copy(src_hbm.at[pl.ds(ba, CH)], src0)
      pltpu.sync_copy(dst_hbm.at[pl.ds(ba, CH)], dst0)
      adjust(src0)
      pltpu.async_copy(tab_hbm.at[src0], rows0, sem0).wait()
      pltpu.sync_copy(rows0, acc.at[dst0], add=True)
    plsc.subcore_barrier()

    for t in range(5):
      pltpu.sync_copy(acc.at[pl.ds(rows_base + 128 * t, 128)], wb)
      pltpu.sync_copy(
          wb, out_hbm.at[pl.ds(c * NPAD + rows_base + 128 * t, 128)])

  k = pl.kernel(
      body,
      out_type=jax.ShapeDtypeStruct((NC * NPAD, dh), jnp.float32),
      mesh=_mesh(),
      scratch_types=[
          pltpu.VMEM((CH,), jnp.int32),
          pltpu.VMEM((CH,), jnp.int32),
          pltpu.VMEM((CH, dh), jnp.float32),
          pltpu.VMEM((CH,), jnp.int32),
          pltpu.VMEM((CH,), jnp.int32),
          pltpu.VMEM((CH, dh), jnp.float32),
          pltpu.VMEM_SHARED((NPAD, dh), jnp.float32),
          pltpu.VMEM((128, dh), jnp.float32),
          pltpu.SemaphoreType.DMA,
          pltpu.SemaphoreType.DMA,
      ],
  )
  return k(tab, src, dst)


def _sc_topk(skey_pad, sc128):
  nv = SG // L

  def body(skey_hbm, sc_hbm, tf_hbm, u_hbm,
           kbuf, keyi, obuf, scm):
    c = lax.axis_index("c")
    s = lax.axis_index("s")
    wid = s * NC + c
    pltpu.sync_copy(sc_hbm, scm)

    iota = lax.iota(jnp.int32, L)

    def scread(idx):
      base = pl.multiple_of((idx // L) * L, 8)
      v = scm[pl.ds(base, L)]
      return jnp.sum(jnp.where(iota == idx - base, v, 0))

    for g_loc in range(2):
      g = wid * 2 + g_loc
      start = scread(g)
      count = scread(B + g)
      a = pl.multiple_of(lax.bitwise_and(start, jnp.int32(-8)), 8)
      off = start - a
      pltpu.sync_copy(skey_hbm.at[pl.ds(a, SG)], kbuf)

      def conv(j, _):
        sl = pl.ds(j * L, L)
        bits = lax.bitcast_convert_type(kbuf[sl], jnp.int32)
        keyi[sl] = _f32key_to_i32(bits)
        return 0
      lax.fori_loop(0, nv, conv, 0)

      offv = jnp.full((L,), off, jnp.int32)
      cntv = jnp.full((L,), count, jnp.int32)
      kq = 4 * count + 4
      kper = lax.shift_right_logical(kq * 52429, 18)

      def count_pred(pred):
        def inner(j, acc):
          sl = pl.ds(j * L, L)
          kv = keyi[sl]
          pos = jnp.full((L,), j * L, jnp.int32) + iota - offv
          valid = (pos >= 0) & (pos < cntv)
          return acc + (pred(kv, pos) & valid).astype(jnp.int32)
        lanes = lax.fori_loop(0, nv, inner, jnp.zeros((L,), jnp.int32))
        return jnp.sum(lanes)

      def bis_a(it, lh):
        lo, hi = lh
        mid = (lax.shift_right_arithmetic(lo, 1)
               + lax.shift_right_arithmetic(hi, 1)
               + (lo & hi & 1))
        ge = count_pred(lambda kv, pos: kv <= mid) >= kper
        return (jnp.where(ge, lo, mid + 1), jnp.where(ge, mid, hi))

      lo, hi = lax.fori_loop(
          0, 32, bis_a,
          (jnp.int32(INT_MIN), jnp.int32(2147483647)))
      t = lo

      strict = count_pred(lambda kv, pos: kv < t)
      r = kper - strict

      def bis_b(it, lh):
        lo2, hi2 = lh
        mid = lax.shift_right_arithmetic(lo2 + hi2, 1)
        ge = count_pred(lambda kv, pos: (kv == t) & (pos < mid)) >= r
        return (jnp.where(ge, lo2, mid + 1), jnp.where(ge, mid, hi2))

      lo2, hi2 = lax.fori_loop(0, 12, bis_b, (jnp.int32(0), count))
      u = start + lo2

      tv = jnp.full((L,), t, jnp.int32)
      tbits = jnp.where(tv >= 0, tv, INT_MIN - tv)
      obuf[0:L] = lax.bitcast_convert_type(tbits, jnp.float32)
      pltpu.sync_copy(obuf, tf_hbm.at[g])
      obuf[0:L] = jnp.full((L,), u.astype(jnp.float32), jnp.float32)
      pltpu.sync_copy(obuf, u_hbm.at[g])

  k = pl.kernel(
      body,
      out_type=(
          jax.ShapeDtypeStruct((B, L), jnp.float32),
          jax.ShapeDtypeStruct((B, L), jnp.float32),
      ),
      mesh=_mesh(),
      scratch_types=[
          pltpu.VMEM((SG,), jnp.float32),
          pltpu.VMEM((SG,), jnp.int32),
          pltpu.VMEM((L,), jnp.float32),
          pltpu.VMEM((2 * B,), jnp.int32),
      ],
      compiler_params=pltpu.CompilerParams(needs_layout_passes=False),
  )
  return k(skey_pad, sc128)


RB = 1000


def _conv_body(weighted, parts, agg_ref, deg_ref, w_ref, xin_ref, wrel_ref,
               b_ref, wroot_ref, batch_ref, h_ref, xsum_ref):
  i = pl.program_id(0)
  d = deg_ref[0, :, 0:1] + deg_ref[1, :, 0:1]
  if weighted:
    w = w_ref[:, 0:1]
    den = jnp.maximum(d * w, 1.0)
  else:
    w = None
    den = jnp.maximum(d, 1.0)

  def scale(a):
    return ((a * w) if weighted else a) / den

  if parts:
    a = scale(agg_ref[0] + agg_ref[1])
    acc = jnp.dot(a, wrel_ref[...], preferred_element_type=jnp.float32, precision=lax.Precision.HIGHEST)
    acc = acc + jnp.dot(xin_ref[...], wroot_ref[...],
                        preferred_element_type=jnp.float32, precision=lax.Precision.HIGHEST)
  else:
    dh = 128
    alo = scale(agg_ref[0])
    ahi = scale(agg_ref[1])
    acc = jnp.dot(alo, wrel_ref[:dh], preferred_element_type=jnp.float32, precision=lax.Precision.HIGHEST)
    acc = acc + jnp.dot(ahi, wrel_ref[dh:], preferred_element_type=jnp.float32, precision=lax.Precision.HIGHEST)
    acc = acc + jnp.dot(xin_ref[0], wroot_ref[:dh],
                        preferred_element_type=jnp.float32, precision=lax.Precision.HIGHEST)
    acc = acc + jnp.dot(xin_ref[1], wroot_ref[dh:],
                        preferred_element_type=jnp.float32, precision=lax.Precision.HIGHEST)
  acc = acc + b_ref[0:1, :]
  h = jnp.maximum(acc, 0.0)
  if weighted:
    h = h * w
  h_ref[0] = h[:, :128]
  h_ref[1] = h[:, 128:]
  gcol = lax.broadcasted_iota(jnp.int32, (RB, B), 1)
  onehot = (batch_ref[:, 0:1] == gcol).astype(jnp.float32)
  part = lax.dot_general(onehot, h, (((0,), (0,)), ((), ())),
                         preferred_element_type=jnp.float32,
                         precision=lax.Precision.HIGHEST)

  @pl.when(i == 0)
  def _():
    xsum_ref[...] = jnp.zeros_like(xsum_ref)

  xsum_ref[...] += part


def _tc_conv(agg2, degsm, w16, xin, wrel, b, wroot, batch2, weighted, parts):
  grid = (N // RB,)
  din = wrel.shape[0]
  dw = degsm.shape[2]
  xin_spec = (pl.BlockSpec((RB, din), lambda i: (i, 0)) if parts
              else pl.BlockSpec((2, RB, 128), lambda i: (0, i, 0)))
  in_specs = [
      pl.BlockSpec((2, RB, 128), lambda i: (0, i, 0)),
      pl.BlockSpec((2, RB, dw), lambda i: (0, i, 0)),
      pl.BlockSpec((RB, 128), lambda i: (i, 0)),
      xin_spec,
      pl.BlockSpec((din, H), lambda i: (0, 0)),
      pl.BlockSpec((1, H), lambda i: (0, 0)),
      pl.BlockSpec((din, H), lambda i: (0, 0)),
      pl.BlockSpec((RB, 1), lambda i: (i, 0)),
  ]
  out_specs = [
      pl.BlockSpec((2, RB, 128), lambda i: (0, i, 0)),
      pl.BlockSpec((B, H), lambda i: (0, 0)),
  ]
  out_shape = [
      jax.ShapeDtypeStruct((2, N, 128), jnp.float32),
      jax.ShapeDtypeStruct((B, H), jnp.float32),
  ]
  return pl.pallas_call(
      functools.partial(_conv_body, weighted, parts),
      grid=grid, in_specs=in_specs, out_specs=out_specs, out_shape=out_shape,
  )(agg2, degsm, w16, xin, wrel, b, wroot, batch2)


def _score_body(h_ref, p_ref, batch_ref, bfull_ref, score_ref, skey_ref,
                sc_ref):
  i = pl.program_id(0)
  p = p_ref[...]
  nrm = jnp.sqrt(jnp.sum(p * p))
  hp = jnp.dot(h_ref[0], p[:128, :], preferred_element_type=jnp.float32, precision=lax.Precision.HIGHEST)
  hp = hp + jnp.dot(h_ref[1], p[128:, :], preferred_element_type=jnp.float32, precision=lax.Precision.HIGHEST)
  s = jnp.tanh(hp / nrm)
  score_ref[...] = s
  skey_ref[...] = batch_ref[...].astype(jnp.float32) * 4.0 - s

  @pl.when(i == 0)
  def _():
    gcol = lax.broadcasted_iota(jnp.int32, (N, B), 1)
    onehot = (bfull_ref[:, 0:1] == gcol).astype(jnp.float32)
    counts = jnp.sum(onehot, axis=0, keepdims=True)
    rr = lax.broadcasted_iota(jnp.int32, (B, B), 0)
    cc = lax.broadcasted_iota(jnp.int32, (B, B), 1)
    tri = (rr < cc).astype(jnp.float32)
    starts = jnp.dot(counts, tri, preferred_element_type=jnp.float32, precision=lax.Precision.HIGHEST)
    sc_ref[...] = jnp.concatenate([starts, counts], axis=1).astype(jnp.int32)


def _tc_score(h2, p0, batch2):
  grid = (N // RB,)
  return pl.pallas_call(
      _score_body,
      grid=grid,
      in_specs=[
          pl.BlockSpec((2, RB, 128), lambda i: (0, i, 0)),
          pl.BlockSpec((H, 1), lambda i: (0, 0)),
          pl.BlockSpec((RB, 1), lambda i: (i, 0)),
          pl.BlockSpec((N, 1), lambda i: (0, 0)),
      ],
      out_specs=[
          pl.BlockSpec((RB, 1), lambda i: (i, 0)),
          pl.BlockSpec((RB, 1), lambda i: (i, 0)),
          pl.BlockSpec((1, 2 * B), lambda i: (0, 0)),
      ],
      out_shape=[
          jax.ShapeDtypeStruct((N, 1), jnp.float32),
          jax.ShapeDtypeStruct((N, 1), jnp.float32),
          jax.ShapeDtypeStruct((1, 2 * B), jnp.int32),
      ],
  )(h2, p0, batch2, batch2)


def _mask_body(h_ref, score_ref, skey_ref, batch_ref, th_ref, h3m_ref,
               m16_ref):
  i = pl.program_id(0)
  gcol = lax.broadcasted_iota(jnp.int32, (RB, B), 1)
  onehot = (batch_ref[:, 0:1] == gcol).astype(jnp.float32)
  g2 = jnp.dot(onehot, th_ref[...], preferred_element_type=jnp.float32, precision=lax.Precision.HIGHEST)
  tf = g2[:, 0:1]
  uu = g2[:, 1:2]
  sk = skey_ref[...]
  idxrow = (lax.broadcasted_iota(jnp.int32, (RB, 1), 0)
            + i * RB).astype(jnp.float32)
  m = ((sk < tf) | ((sk == tf) & (idxrow < uu))).astype(jnp.float32)
  hm = score_ref[...] * m
  h3m_ref[0] = h_ref[0] * hm
  h3m_ref[1] = h_ref[1] * hm
  m16_ref[...] = jnp.broadcast_to(m, (RB, 128))


def _tc_mask(h2, score, skey, batch2, th3):
  grid = (N // RB,)
  return pl.pallas_call(
      _mask_body,
      grid=grid,
      in_specs=[
          pl.BlockSpec((2, RB, 128), lambda i: (0, i, 0)),
          pl.BlockSpec((RB, 1), lambda i: (i, 0)),
          pl.BlockSpec((RB, 1), lambda i: (i, 0)),
          pl.BlockSpec((RB, 1), lambda i: (i, 0)),
          pl.BlockSpec((B, 2), lambda i: (0, 0)),
      ],
      out_specs=[
          pl.BlockSpec((2, RB, 128), lambda i: (0, i, 0)),
          pl.BlockSpec((RB, 128), lambda i: (i, 0)),
      ],
      out_shape=[
          jax.ShapeDtypeStruct((2, N, 128), jnp.float32),
          jax.ShapeDtypeStruct((N, 128), jnp.float32),
      ],
  )(h2, score, skey, batch2, th3)


def _mlp_body(x0, x1, x2, x3, w1, b1, w2, b2, w3, b3, out_ref):
  z = jnp.dot(x0[...], w1[:H], preferred_element_type=jnp.float32, precision=lax.Precision.HIGHEST)
  z = z + jnp.dot(x1[...], w1[H:2 * H], preferred_element_type=jnp.float32, precision=lax.Precision.HIGHEST)
  z = z + jnp.dot(x2[...], w1[2 * H:3 * H], preferred_element_type=jnp.float32, precision=lax.Precision.HIGHEST)
  z = z + jnp.dot(x3[...], w1[3 * H:], preferred_element_type=jnp.float32, precision=lax.Precision.HIGHEST)
  z = jnp.maximum(z + b1[0:1, :], 0.0)
  z = jnp.maximum(jnp.dot(z, w2[...], preferred_element_type=jnp.float32, precision=lax.Precision.HIGHEST)
                  + b2[0:1, :], 0.0)
  z = jnp.dot(z, w3[...], preferred_element_type=jnp.float32, precision=lax.Precision.HIGHEST) + b3[0:1, :]
  mx = jnp.max(z, axis=1, keepdims=True)
  sh = z - mx
  out_ref[...] = sh - jnp.log(jnp.sum(jnp.exp(sh), axis=1, keepdims=True))


def _tc_mlp(xs, params):
  return pl.pallas_call(
      _mlp_body,
      out_shape=jax.ShapeDtypeStruct((B, C), jnp.float32),
  )(xs[0], xs[1], xs[2], xs[3],
    params['W1'], params['b1'].reshape(1, H),
    params['W2'], params['b2'].reshape(1, H // 2),
    params['W3'], params['b3'].reshape(1, C))


def kernel(x, params, edge_index, batch):
  src = edge_index[0]
  dst = edge_index[1]
  batch2 = batch.reshape(N, 1)

  ones128 = jnp.ones((CH, 128), jnp.float32)
  zrs128 = jnp.zeros((128, 128), jnp.float32)
  deg = _sc_degree(ones128, zrs128, dst).reshape(2, NPAD, 128)

  agg1 = _sc_segsum_feat(x, src, dst, False).reshape(2, NPAD, 128)
  h1, xs0 = _tc_conv(agg1, deg, x, x, params['W_rel1'],
                     params['b_rel1'].reshape(1, H), params['W_root1'],
                     batch2, False, True)

  h1_flat = h1.reshape(2 * N, 128)
  agg2 = _sc_segsum_feat(h1_flat, src, dst, True).reshape(2, NPAD, 128)
  h2, xs1 = _tc_conv(agg2, deg, agg2[0, :N], h1, params['W_rel2'],
                     params['b_rel2'].reshape(1, H), params['W_root2'],
                     batch2, False, False)

  score, skey, sc128 = _tc_score(h2, params['p0'].reshape(H, 1), batch2)
  skey_pad = jnp.concatenate(
      [skey.reshape(N), jnp.full((SKEY_PAD,), 1e30, jnp.float32)])
  tf16, u16 = _sc_topk(skey_pad, sc128.reshape(2 * B))
  th2 = jnp.concatenate([tf16[:, 0:1], u16[:, 0:1]], axis=1)

  h3m, m16 = _tc_mask(h2, score, skey, batch2, th2)

  sm = _sc_segsum_feat(m16, src, dst, False).reshape(2, NPAD, 128)

  h3m_flat = h3m.reshape(2 * N, 128)
  agg3 = _sc_segsum_feat(h3m_flat, src, dst, True).reshape(2, NPAD, 128)
  h4m, xs2 = _tc_conv(agg3, sm, m16, h3m, params['W_rel3'],
                      params['b_rel3'].reshape(1, H), params['W_root3'],
                      batch2, True, False)

  h4m_flat = h4m.reshape(2 * N, 128)
  agg4 = _sc_segsum_feat(h4m_flat, src, dst, True).reshape(2, NPAD, 128)
  _, xs3 = _tc_conv(agg4, sm, m16, h4m, params['W_rel4'],
                    params['b_rel4'].reshape(1, H), params['W_root4'],
                    batch2, True, False)

  return _tc_mlp([xs0, xs1, xs2, xs3], params)


# speedup vs baseline: 8.0167x; 1.2758x over previous
import functools

import jax
import jax.numpy as jnp
from jax import lax
from jax.experimental import pallas as pl
from jax.experimental.pallas import tpu as pltpu
from jax.experimental.pallas import tpu_sc as plsc

N, E, F, H, B, C = 10000, 320000, 128, 256, 64, 10

NC, NS, L = 2, 16, 16
NPAD = 10240
CH = 80
SKEY_PAD = 1024
SG = 544
INT_MIN = -2147483648

_mesh = functools.partial(
    plsc.VectorSubcoreMesh, core_axis_name="c", subcore_axis_name="s",
    num_cores=NC, num_subcores=NS)


def _f32key_to_i32(b):
  return jnp.where(b >= 0, b, INT_MIN - b)


def _sc_degree(ones128, zrs128, dst):
  dh = 128
  nch = E // (NC * NS * CH)

  def body(ones_hbm, zrs_hbm, dst_hbm, out_hbm, ones_v, dstb, d0, acc, wb):
    c = lax.axis_index("c")
    s = lax.axis_index("s")
    rows0 = s * (NPAD // NS)

    pltpu.sync_copy(ones_hbm, ones_v)
    pltpu.sync_copy(zrs_hbm, wb)
    for t in range(5):
      pltpu.sync_copy(wb, acc.at[pl.ds(rows0 + 128 * t, 128)])
    base0 = (s * NC + c) * nch * CH
    pltpu.sync_copy(dst_hbm.at[pl.ds(base0, nch * CH)], dstb)
    plsc.subcore_barrier()

    def chunk(j, _):
      for kk in range(CH // L):
        sl = pl.multiple_of(j * CH + kk * L, 8)
        d0[pl.ds(kk * L, L)] = dstb[pl.ds(sl, L)]
      pltpu.sync_copy(ones_v, acc.at[d0], add=True)
      return 0
    lax.fori_loop(0, nch, chunk, 0)
    plsc.subcore_barrier()

    for t in range(5):
      pltpu.sync_copy(acc.at[pl.ds(rows0 + 128 * t, 128)], wb)
      pltpu.sync_copy(wb, out_hbm.at[pl.ds(c * NPAD + rows0 + 128 * t, 128)])

  k = pl.kernel(
      body,
      out_type=jax.ShapeDtypeStruct((NC * NPAD, dh), jnp.float32),
      mesh=_mesh(),
      scratch_types=[
          pltpu.VMEM((CH, dh), jnp.float32),
          pltpu.VMEM((E // (NC * NS),), jnp.int32),
          pltpu.VMEM((CH,), jnp.int32),
          pltpu.VMEM_SHARED((NPAD, dh), jnp.float32),
          pltpu.VMEM((128, dh), jnp.float32),
      ],
  )
  return k(ones128, zrs128, dst)


def _sc_segsum_feat(tab, src, dst, feat_split):
  dh = 128
  pw = E // NS if feat_split else E // (NC * NS)
  nch = pw // CH
  G = 25
  nblk = nch // G
  assert nblk * G == nch

  def body(tab_hbm, src_hbm, dst_hbm, out_hbm,
           srcb, dstb, g0, g1, d0, d1, rows0, rows1, acc, sem0, sem1):
    c = lax.axis_index("c")
    s = lax.axis_index("s")
    rows_base = s * (NPAD // NS)

    def zrow(i, _):
      def zlane(k, __):
        rows0[i, pl.ds(k * L, L)] = jnp.zeros((L,), jnp.float32)
        return 0
      lax.fori_loop(0, dh // L, zlane, 0)
      return 0
    lax.fori_loop(0, CH, zrow, 0)
    for t in range(8):
      pltpu.sync_copy(rows0, acc.at[pl.ds(rows_base + CH * t, CH)])

    if feat_split:
      base0 = s * pw
      coff = c * N
    else:
      base0 = (s * NC + c) * pw
      coff = None
    plsc.subcore_barrier()

    offv = (jnp.full((L,), coff, jnp.int32) if coff is not None
            else jnp.zeros((L,), jnp.int32))

    def load_idx(jj, gb, db):
      for kk in range(CH // L):
        sl = pl.multiple_of(jj * CH + kk * L, 8)
        gb[pl.ds(kk * L, L)] = srcb[pl.ds(sl, L)] + offv
        db[pl.ds(kk * L, L)] = dstb[pl.ds(sl, L)]

    def blk(bi, _):
      boff = base0 + bi * (G * CH)
      pltpu.sync_copy(src_hbm.at[pl.ds(boff, G * CH)], srcb)
      pltpu.sync_copy(dst_hbm.at[pl.ds(boff, G * CH)], dstb)

      def pair(j, _):
        ja = 2 * j
        jb = 2 * j + 1
        load_idx(ja, g0, d0)
        cp0 = pltpu.async_copy(tab_hbm.at[g0], rows0, sem0)
        load_idx(jb, g1, d1)
        cp1 = pltpu.async_copy(tab_hbm.at[g1], rows1, sem1)
        cp0.wait()
        pltpu.sync_copy(rows0, acc.at[d0], add=True)
        cp1.wait()
        pltpu.sync_copy(rows1, acc.at[d1], add=True)
        return 0
      lax.fori_loop(0, G // 2, pair, 0)
      load_idx(G - 1, g0, d0)
      pltpu.async_copy(tab_hbm.at[g0], rows0, sem0).wait()
      pltpu.sync_copy(rows0, acc.at[d0], add=True)
      return 0
    lax.fori_loop(0, nblk, blk, 0)
    plsc.subcore_barrier()

    for t in range(8):
      pltpu.sync_copy(acc.at[pl.ds(rows_base + CH * t, CH)], rows0)
      pltpu.sync_copy(
          rows0, out_hbm.at[pl.ds(c * NPAD + rows_base + CH * t, CH)])

  k = pl.kernel(
      body,
      out_type=jax.ShapeDtypeStruct((NC * NPAD, dh), jnp.float32),
      mesh=_mesh(),
      scratch_types=(
          [pltpu.VMEM((G * CH,), jnp.int32)] * 2
          + [pltpu.VMEM((CH,), jnp.int32)] * 4
          + [pltpu.VMEM((CH, dh), jnp.float32)] * 2
          + [pltpu.VMEM_SHARED((NPAD, dh), jnp.float32)]
          + [pltpu.SemaphoreType.DMA] * 2
      ),
  )
  return k(tab, src, dst)


def _sc_topk(skey_pad, sc128):
  nv = SG // L

  def body(skey_hbm, sc_hbm, tf_hbm, u_hbm,
           kbuf, keyi, obuf, scm):
    c = lax.axis_index("c")
    s = lax.axis_index("s")
    wid = s * NC + c
    pltpu.sync_copy(sc_hbm, scm)

    iota = lax.iota(jnp.int32, L)

    def scread(idx):
      base = pl.multiple_of((idx // L) * L, 8)
      v = scm[pl.ds(base, L)]
      return jnp.sum(jnp.where(iota == idx - base, v, 0))

    for g_loc in range(2):
      g = wid * 2 + g_loc
      start = scread(g)
      count = scread(B + g)
      a = pl.multiple_of(lax.bitwise_and(start, jnp.int32(-8)), 8)
      off = start - a
      pltpu.sync_copy(skey_hbm.at[pl.ds(a, SG)], kbuf)

      def conv(j, _):
        sl = pl.ds(j * L, L)
        bits = lax.bitcast_convert_type(kbuf[sl], jnp.int32)
        keyi[sl] = _f32key_to_i32(bits)
        return 0
      lax.fori_loop(0, nv, conv, 0)

      offv = jnp.full((L,), off, jnp.int32)
      cntv = jnp.full((L,), count, jnp.int32)
      kq = 4 * count + 4
      kper = lax.shift_right_logical(kq * 52429, 18)

      def count_pred(pred):
        def inner(j, acc):
          sl = pl.ds(j * L, L)
          kv = keyi[sl]
          pos = jnp.full((L,), j * L, jnp.int32) + iota - offv
          valid = (pos >= 0) & (pos < cntv)
          return acc + (pred(kv, pos) & valid).astype(jnp.int32)
        lanes = lax.fori_loop(0, nv, inner, jnp.zeros((L,), jnp.int32))
        return jnp.sum(lanes)

      def bis_a(it, lh):
        lo, hi = lh
        mid = (lax.shift_right_arithmetic(lo, 1)
               + lax.shift_right_arithmetic(hi, 1)
               + (lo & hi & 1))
        ge = count_pred(lambda kv, pos: kv <= mid) >= kper
        return (jnp.where(ge, lo, mid + 1), jnp.where(ge, mid, hi))

      lo, hi = lax.fori_loop(
          0, 32, bis_a,
          (jnp.int32(INT_MIN), jnp.int32(2147483647)))
      t = lo

      strict = count_pred(lambda kv, pos: kv < t)
      r = kper - strict

      def bis_b(it, lh):
        lo2, hi2 = lh
        mid = lax.shift_right_arithmetic(lo2 + hi2, 1)
        ge = count_pred(lambda kv, pos: (kv == t) & (pos < mid)) >= r
        return (jnp.where(ge, lo2, mid + 1), jnp.where(ge, mid, hi2))

      lo2, hi2 = lax.fori_loop(0, 12, bis_b, (jnp.int32(0), count))
      u = start + lo2

      tv = jnp.full((L,), t, jnp.int32)
      tbits = jnp.where(tv >= 0, tv, INT_MIN - tv)
      obuf[0:L] = lax.bitcast_convert_type(tbits, jnp.float32)
      pltpu.sync_copy(obuf, tf_hbm.at[g])
      obuf[0:L] = jnp.full((L,), u.astype(jnp.float32), jnp.float32)
      pltpu.sync_copy(obuf, u_hbm.at[g])

  k = pl.kernel(
      body,
      out_type=(
          jax.ShapeDtypeStruct((B, L), jnp.float32),
          jax.ShapeDtypeStruct((B, L), jnp.float32),
      ),
      mesh=_mesh(),
      scratch_types=[
          pltpu.VMEM((SG,), jnp.float32),
          pltpu.VMEM((SG,), jnp.int32),
          pltpu.VMEM((L,), jnp.float32),
          pltpu.VMEM((2 * B,), jnp.int32),
      ],
      compiler_params=pltpu.CompilerParams(needs_layout_passes=False),
  )
  return k(skey_pad, sc128)


RB = 1000


def _conv_body(weighted, parts, agg_ref, deg_ref, w_ref, xin_ref, wrel_ref,
               b_ref, wroot_ref, batch_ref, h_ref, xsum_ref):
  i = pl.program_id(0)
  d = deg_ref[0, :, 0:1] + deg_ref[1, :, 0:1]
  if weighted:
    w = w_ref[:, 0:1]
    den = jnp.maximum(d * w, 1.0)
  else:
    w = None
    den = jnp.maximum(d, 1.0)

  def scale(a):
    return ((a * w) if weighted else a) / den

  if parts:
    a = scale(agg_ref[0] + agg_ref[1])
    acc = jnp.dot(a, wrel_ref[...], preferred_element_type=jnp.float32, precision=lax.Precision.HIGHEST)
    acc = acc + jnp.dot(xin_ref[...], wroot_ref[...],
                        preferred_element_type=jnp.float32, precision=lax.Precision.HIGHEST)
  else:
    dh = 128
    alo = scale(agg_ref[0])
    ahi = scale(agg_ref[1])
    acc = jnp.dot(alo, wrel_ref[:dh], preferred_element_type=jnp.float32, precision=lax.Precision.HIGHEST)
    acc = acc + jnp.dot(ahi, wrel_ref[dh:], preferred_element_type=jnp.float32, precision=lax.Precision.HIGHEST)
    acc = acc + jnp.dot(xin_ref[0], wroot_ref[:dh],
                        preferred_element_type=jnp.float32, precision=lax.Precision.HIGHEST)
    acc = acc + jnp.dot(xin_ref[1], wroot_ref[dh:],
                        preferred_element_type=jnp.float32, precision=lax.Precision.HIGHEST)
  acc = acc + b_ref[0:1, :]
  h = jnp.maximum(acc, 0.0)
  if weighted:
    h = h * w
  h_ref[0] = h[:, :128]
  h_ref[1] = h[:, 128:]
  gcol = lax.broadcasted_iota(jnp.int32, (RB, B), 1)
  onehot = (batch_ref[:, 0:1] == gcol).astype(jnp.float32)
  part = lax.dot_general(onehot, h, (((0,), (0,)), ((), ())),
                         preferred_element_type=jnp.float32,
                         precision=lax.Precision.HIGHEST)

  @pl.when(i == 0)
  def _():
    xsum_ref[...] = jnp.zeros_like(xsum_ref)

  xsum_ref[...] += part


def _tc_conv(agg2, degsm, w16, xin, wrel, b, wroot, batch2, weighted, parts):
  grid = (N // RB,)
  din = wrel.shape[0]
  dw = degsm.shape[2]
  xin_spec = (pl.BlockSpec((RB, din), lambda i: (i, 0)) if parts
              else pl.BlockSpec((2, RB, 128), lambda i: (0, i, 0)))
  in_specs = [
      pl.BlockSpec((2, RB, 128), lambda i: (0, i, 0)),
      pl.BlockSpec((2, RB, dw), lambda i: (0, i, 0)),
      pl.BlockSpec((RB, 128), lambda i: (i, 0)),
      xin_spec,
      pl.BlockSpec((din, H), lambda i: (0, 0)),
      pl.BlockSpec((1, H), lambda i: (0, 0)),
      pl.BlockSpec((din, H), lambda i: (0, 0)),
      pl.BlockSpec((RB, 1), lambda i: (i, 0)),
  ]
  out_specs = [
      pl.BlockSpec((2, RB, 128), lambda i: (0, i, 0)),
      pl.BlockSpec((B, H), lambda i: (0, 0)),
  ]
  out_shape = [
      jax.ShapeDtypeStruct((2, N, 128), jnp.float32),
      jax.ShapeDtypeStruct((B, H), jnp.float32),
  ]
  return pl.pallas_call(
      functools.partial(_conv_body, weighted, parts),
      grid=grid, in_specs=in_specs, out_specs=out_specs, out_shape=out_shape,
  )(agg2, degsm, w16, xin, wrel, b, wroot, batch2)


def _score_body(h_ref, p_ref, batch_ref, bfull_ref, score_ref, skey_ref,
                sc_ref):
  i = pl.program_id(0)
  p = p_ref[...]
  nrm = jnp.sqrt(jnp.sum(p * p))
  hp = jnp.dot(h_ref[0], p[:128, :], preferred_element_type=jnp.float32, precision=lax.Precision.HIGHEST)
  hp = hp + jnp.dot(h_ref[1], p[128:, :], preferred_element_type=jnp.float32, precision=lax.Precision.HIGHEST)
  s = jnp.tanh(hp / nrm)
  score_ref[...] = s
  skey_ref[...] = batch_ref[...].astype(jnp.float32) * 4.0 - s

  @pl.when(i == 0)
  def _():
    gcol = lax.broadcasted_iota(jnp.int32, (N, B), 1)
    onehot = (bfull_ref[:, 0:1] == gcol).astype(jnp.float32)
    counts = jnp.sum(onehot, axis=0, keepdims=True)
    rr = lax.broadcasted_iota(jnp.int32, (B, B), 0)
    cc = lax.broadcasted_iota(jnp.int32, (B, B), 1)
    tri = (rr < cc).astype(jnp.float32)
    starts = jnp.dot(counts, tri, preferred_element_type=jnp.float32, precision=lax.Precision.HIGHEST)
    sc_ref[...] = jnp.concatenate([starts, counts], axis=1).astype(jnp.int32)


def _tc_score(h2, p0, batch2):
  grid = (N // RB,)
  return pl.pallas_call(
      _score_body,
      grid=grid,
      in_specs=[
          pl.BlockSpec((2, RB, 128), lambda i: (0, i, 0)),
          pl.BlockSpec((H, 1), lambda i: (0, 0)),
          pl.BlockSpec((RB, 1), lambda i: (i, 0)),
          pl.BlockSpec((N, 1), lambda i: (0, 0)),
      ],
      out_specs=[
          pl.BlockSpec((RB, 1), lambda i: (i, 0)),
          pl.BlockSpec((RB, 1), lambda i: (i, 0)),
          pl.BlockSpec((1, 2 * B), lambda i: (0, 0)),
      ],
      out_shape=[
          jax.ShapeDtypeStruct((N, 1), jnp.float32),
          jax.ShapeDtypeStruct((N, 1), jnp.float32),
          jax.ShapeDtypeStruct((1, 2 * B), jnp.int32),
      ],
  )(h2, p0, batch2, batch2)


def _mask_body(h_ref, score_ref, skey_ref, batch_ref, th_ref, h3m_ref,
               m16_ref):
  i = pl.program_id(0)
  gcol = lax.broadcasted_iota(jnp.int32, (RB, B), 1)
  onehot = (batch_ref[:, 0:1] == gcol).astype(jnp.float32)
  g2 = jnp.dot(onehot, th_ref[...], preferred_element_type=jnp.float32, precision=lax.Precision.HIGHEST)
  tf = g2[:, 0:1]
  uu = g2[:, 1:2]
  sk = skey_ref[...]
  idxrow = (lax.broadcasted_iota(jnp.int32, (RB, 1), 0)
            + i * RB).astype(jnp.float32)
  m = ((sk < tf) | ((sk == tf) & (idxrow < uu))).astype(jnp.float32)
  hm = score_ref[...] * m
  h3m_ref[0] = h_ref[0] * hm
  h3m_ref[1] = h_ref[1] * hm
  m16_ref[...] = jnp.broadcast_to(m, (RB, 128))


def _tc_mask(h2, score, skey, batch2, th3):
  grid = (N // RB,)
  return pl.pallas_call(
      _mask_body,
      grid=grid,
      in_specs=[
          pl.BlockSpec((2, RB, 128), lambda i: (0, i, 0)),
          pl.BlockSpec((RB, 1), lambda i: (i, 0)),
          pl.BlockSpec((RB, 1), lambda i: (i, 0)),
          pl.BlockSpec((RB, 1), lambda i: (i, 0)),
          pl.BlockSpec((B, 2), lambda i: (0, 0)),
      ],
      out_specs=[
          pl.BlockSpec((2, RB, 128), lambda i: (0, i, 0)),
          pl.BlockSpec((RB, 128), lambda i: (i, 0)),
      ],
      out_shape=[
          jax.ShapeDtypeStruct((2, N, 128), jnp.float32),
          jax.ShapeDtypeStruct((N, 128), jnp.float32),
      ],
  )(h2, score, skey, batch2, th3)


def _mlp_body(x0, x1, x2, x3, w1, b1, w2, b2, w3, b3, out_ref):
  z = jnp.dot(x0[...], w1[:H], preferred_element_type=jnp.float32, precision=lax.Precision.HIGHEST)
  z = z + jnp.dot(x1[...], w1[H:2 * H], preferred_element_type=jnp.float32, precision=lax.Precision.HIGHEST)
  z = z + jnp.dot(x2[...], w1[2 * H:3 * H], preferred_element_type=jnp.float32, precision=lax.Precision.HIGHEST)
  z = z + jnp.dot(x3[...], w1[3 * H:], preferred_element_type=jnp.float32, precision=lax.Precision.HIGHEST)
  z = jnp.maximum(z + b1[0:1, :], 0.0)
  z = jnp.maximum(jnp.dot(z, w2[...], preferred_element_type=jnp.float32, precision=lax.Precision.HIGHEST)
                  + b2[0:1, :], 0.0)
  z = jnp.dot(z, w3[...], preferred_element_type=jnp.float32, precision=lax.Precision.HIGHEST) + b3[0:1, :]
  mx = jnp.max(z, axis=1, keepdims=True)
  sh = z - mx
  out_ref[...] = sh - jnp.log(jnp.sum(jnp.exp(sh), axis=1, keepdims=True))


def _tc_mlp(xs, params):
  return pl.pallas_call(
      _mlp_body,
      out_shape=jax.ShapeDtypeStruct((B, C), jnp.float32),
  )(xs[0], xs[1], xs[2], xs[3],
    params['W1'], params['b1'].reshape(1, H),
    params['W2'], params['b2'].reshape(1, H // 2),
    params['W3'], params['b3'].reshape(1, C))


def kernel(x, params, edge_index, batch):
  src = edge_index[0]
  dst = edge_index[1]
  batch2 = batch.reshape(N, 1)

  ones128 = jnp.ones((CH, 128), jnp.float32)
  zrs128 = jnp.zeros((128, 128), jnp.float32)
  deg = _sc_degree(ones128, zrs128, dst).reshape(2, NPAD, 128)

  agg1 = _sc_segsum_feat(x, src, dst, False).reshape(2, NPAD, 128)
  h1, xs0 = _tc_conv(agg1, deg, x, x, params['W_rel1'],
                     params['b_rel1'].reshape(1, H), params['W_root1'],
                     batch2, False, True)

  h1_flat = h1.reshape(2 * N, 128)
  agg2 = _sc_segsum_feat(h1_flat, src, dst, True).reshape(2, NPAD, 128)
  h2, xs1 = _tc_conv(agg2, deg, agg2[0, :N], h1, params['W_rel2'],
                     params['b_rel2'].reshape(1, H), params['W_root2'],
                     batch2, False, False)

  score, skey, sc128 = _tc_score(h2, params['p0'].reshape(H, 1), batch2)
  skey_pad = jnp.concatenate(
      [skey.reshape(N), jnp.full((SKEY_PAD,), 1e30, jnp.float32)])
  tf16, u16 = _sc_topk(skey_pad, sc128.reshape(2 * B))
  th2 = jnp.concatenate([tf16[:, 0:1], u16[:, 0:1]], axis=1)

  h3m, m16 = _tc_mask(h2, score, skey, batch2, th2)

  sm = _sc_segsum_feat(m16, src, dst, False).reshape(2, NPAD, 128)

  h3m_flat = h3m.reshape(2 * N, 128)
  agg3 = _sc_segsum_feat(h3m_flat, src, dst, True).reshape(2, NPAD, 128)
  h4m, xs2 = _tc_conv(agg3, sm, m16, h3m, params['W_rel3'],
                      params['b_rel3'].reshape(1, H), params['W_root3'],
                      batch2, True, False)

  h4m_flat = h4m.reshape(2 * N, 128)
  agg4 = _sc_segsum_feat(h4m_flat, src, dst, True).reshape(2, NPAD, 128)
  _, xs3 = _tc_conv(agg4, sm, m16, h4m, params['W_rel4'],
                    params['b_rel4'].reshape(1, H), params['W_root4'],
                    batch2, True, False)

  return _tc_mlp([xs0, xs1, xs2, xs3], params)


# speedup vs baseline: 8.8670x; 1.1061x over previous
import functools

import jax
import jax.numpy as jnp
from jax import lax
from jax.experimental import pallas as pl
from jax.experimental.pallas import tpu as pltpu
from jax.experimental.pallas import tpu_sc as plsc

N, E, F, H, B, C = 10000, 320000, 128, 256, 64, 10

NC, NS, L = 2, 16, 16
NPAD = 10240
CH = 80
SKEY_PAD = 1024
SG = 544
INT_MIN = -2147483648

_mesh = functools.partial(
    plsc.VectorSubcoreMesh, core_axis_name="c", subcore_axis_name="s",
    num_cores=NC, num_subcores=NS)


def _f32key_to_i32(b):
  return jnp.where(b >= 0, b, INT_MIN - b)


def _sc_degree(ones128, zrs128, dst):
  dh = 128
  nch = E // (NC * NS * CH)

  def body(ones_hbm, zrs_hbm, dst_hbm, out_hbm, ones_v, dstb, d0, acc, wb):
    c = lax.axis_index("c")
    s = lax.axis_index("s")
    rows0 = s * (NPAD // NS)

    pltpu.sync_copy(ones_hbm, ones_v)
    pltpu.sync_copy(zrs_hbm, wb)
    for t in range(5):
      pltpu.sync_copy(wb, acc.at[pl.ds(rows0 + 128 * t, 128)])
    base0 = (s * NC + c) * nch * CH
    pltpu.sync_copy(dst_hbm.at[pl.ds(base0, nch * CH)], dstb)
    plsc.subcore_barrier()

    def chunk(j, _):
      for kk in range(CH // L):
        sl = pl.multiple_of(j * CH + kk * L, 8)
        d0[pl.ds(kk * L, L)] = dstb[pl.ds(sl, L)]
      pltpu.sync_copy(ones_v, acc.at[d0], add=True)
      return 0
    lax.fori_loop(0, nch, chunk, 0)
    plsc.subcore_barrier()

    for t in range(5):
      pltpu.sync_copy(acc.at[pl.ds(rows0 + 128 * t, 128)], wb)
      pltpu.sync_copy(wb, out_hbm.at[pl.ds(c * NPAD + rows0 + 128 * t, 128)])

  k = pl.kernel(
      body,
      out_type=jax.ShapeDtypeStruct((NC * NPAD, dh), jnp.float32),
      mesh=_mesh(),
      scratch_types=[
          pltpu.VMEM((CH, dh), jnp.float32),
          pltpu.VMEM((E // (NC * NS),), jnp.int32),
          pltpu.VMEM((CH,), jnp.int32),
          pltpu.VMEM_SHARED((NPAD, dh), jnp.float32),
          pltpu.VMEM((128, dh), jnp.float32),
      ],
  )
  return k(ones128, zrs128, dst)


def _sc_segsum_feat(tab, src, dst, feat_split):
  dh = 128
  pw = E // NS if feat_split else E // (NC * NS)
  nch = pw // CH
  G = 25
  nblk = nch // G
  assert nblk * G == nch

  def body(tab_hbm, src_hbm, dst_hbm, out_hbm,
           srcb, dstb, g0, g1, g2, g3, d0, d1, d2, d3,
           rows0, rows1, rows2, rows3, acc,
           sem0, sem1, sem2, sem3, ssem0, ssem1, ssem2, ssem3):
    c = lax.axis_index("c")
    s = lax.axis_index("s")
    rows_base = s * (NPAD // NS)

    def zrow(i, _):
      def zlane(k, __):
        rows0[i, pl.ds(k * L, L)] = jnp.zeros((L,), jnp.float32)
        return 0
      lax.fori_loop(0, dh // L, zlane, 0)
      return 0
    lax.fori_loop(0, CH, zrow, 0)
    for t in range(8):
      pltpu.sync_copy(rows0, acc.at[pl.ds(rows_base + CH * t, CH)])

    if feat_split:
      base0 = s * pw
      coff = c * N
    else:
      base0 = (s * NC + c) * pw
      coff = None
    plsc.subcore_barrier()

    offv = (jnp.full((L,), coff, jnp.int32) if coff is not None
            else jnp.zeros((L,), jnp.int32))

    def load_idx(jj, gb, db):
      for kk in range(CH // L):
        sl = pl.multiple_of(jj * CH + kk * L, 8)
        gb[pl.ds(kk * L, L)] = srcb[pl.ds(sl, L)] + offv
        db[pl.ds(kk * L, L)] = dstb[pl.ds(sl, L)]

    def blk(bi, _):
      boff = base0 + bi * (G * CH)
      pltpu.sync_copy(src_hbm.at[pl.ds(boff, G * CH)], srcb)
      pltpu.sync_copy(dst_hbm.at[pl.ds(boff, G * CH)], dstb)

      def quad(q, _):
        j0 = 4 * q
        load_idx(j0 + 0, g0, d0)
        cpa = pltpu.async_copy(tab_hbm.at[g0], rows0, sem0)
        load_idx(j0 + 1, g1, d1)
        cpb = pltpu.async_copy(tab_hbm.at[g1], rows1, sem1)
        load_idx(j0 + 2, g2, d2)
        cpc = pltpu.async_copy(tab_hbm.at[g2], rows2, sem2)
        load_idx(j0 + 3, g3, d3)
        cpd = pltpu.async_copy(tab_hbm.at[g3], rows3, sem3)
        cpa.wait()
        sca = pltpu.async_copy(rows0, acc.at[d0], ssem0, add=True)
        cpb.wait()
        scb = pltpu.async_copy(rows1, acc.at[d1], ssem1, add=True)
        cpc.wait()
        scc = pltpu.async_copy(rows2, acc.at[d2], ssem2, add=True)
        cpd.wait()
        scd = pltpu.async_copy(rows3, acc.at[d3], ssem3, add=True)
        sca.wait()
        scb.wait()
        scc.wait()
        scd.wait()
        return 0
      lax.fori_loop(0, G // 4, quad, 0)
      load_idx(G - 1, g0, d0)
      pltpu.async_copy(tab_hbm.at[g0], rows0, sem0).wait()
      pltpu.sync_copy(rows0, acc.at[d0], add=True)
      return 0
    lax.fori_loop(0, nblk, blk, 0)
    plsc.subcore_barrier()

    for t in range(8):
      pltpu.sync_copy(acc.at[pl.ds(rows_base + CH * t, CH)], rows0)
      pltpu.sync_copy(
          rows0, out_hbm.at[pl.ds(c * NPAD + rows_base + CH * t, CH)])

  k = pl.kernel(
      body,
      out_type=jax.ShapeDtypeStruct((NC * NPAD, dh), jnp.float32),
      mesh=_mesh(),
      scratch_types=(
          [pltpu.VMEM((G * CH,), jnp.int32)] * 2
          + [pltpu.VMEM((CH,), jnp.int32)] * 8
          + [pltpu.VMEM((CH, dh), jnp.float32)] * 4
          + [pltpu.VMEM_SHARED((NPAD, dh), jnp.float32)]
          + [pltpu.SemaphoreType.DMA] * 8
      ),
  )
  return k(tab, src, dst)


def _sc_topk(skey_pad, sc128):
  nv = SG // L

  def body(skey_hbm, sc_hbm, tf_hbm, u_hbm,
           kbuf, keyi, obuf, scm):
    c = lax.axis_index("c")
    s = lax.axis_index("s")
    wid = s * NC + c
    pltpu.sync_copy(sc_hbm, scm)

    iota = lax.iota(jnp.int32, L)

    def scread(idx):
      base = pl.multiple_of((idx // L) * L, 8)
      v = scm[pl.ds(base, L)]
      return jnp.sum(jnp.where(iota == idx - base, v, 0))

    for g_loc in range(2):
      g = wid * 2 + g_loc
      start = scread(g)
      count = scread(B + g)
      a = pl.multiple_of(lax.bitwise_and(start, jnp.int32(-8)), 8)
      off = start - a
      pltpu.sync_copy(skey_hbm.at[pl.ds(a, SG)], kbuf)

      def conv(j, _):
        sl = pl.ds(j * L, L)
        bits = lax.bitcast_convert_type(kbuf[sl], jnp.int32)
        keyi[sl] = _f32key_to_i32(bits)
        return 0
      lax.fori_loop(0, nv, conv, 0)

      offv = jnp.full((L,), off, jnp.int32)
      cntv = jnp.full((L,), count, jnp.int32)
      kq = 4 * count + 4
      kper = lax.shift_right_logical(kq * 52429, 18)

      def count_pred(pred):
        def inner(j, acc):
          sl = pl.ds(j * L, L)
          kv = keyi[sl]
          pos = jnp.full((L,), j * L, jnp.int32) + iota - offv
          valid = (pos >= 0) & (pos < cntv)
          return acc + (pred(kv, pos) & valid).astype(jnp.int32)
        lanes = lax.fori_loop(0, nv, inner, jnp.zeros((L,), jnp.int32))
        return jnp.sum(lanes)

      def bis_a(it, lh):
        lo, hi = lh
        mid = (lax.shift_right_arithmetic(lo, 1)
               + lax.shift_right_arithmetic(hi, 1)
               + (lo & hi & 1))
        ge = count_pred(lambda kv, pos: kv <= mid) >= kper
        return (jnp.where(ge, lo, mid + 1), jnp.where(ge, mid, hi))

      lo, hi = lax.fori_loop(
          0, 32, bis_a,
          (jnp.int32(INT_MIN), jnp.int32(2147483647)))
      t = lo

      strict = count_pred(lambda kv, pos: kv < t)
      r = kper - strict

      def bis_b(it, lh):
        lo2, hi2 = lh
        mid = lax.shift_right_arithmetic(lo2 + hi2, 1)
        ge = count_pred(lambda kv, pos: (kv == t) & (pos < mid)) >= r
        return (jnp.where(ge, lo2, mid + 1), jnp.where(ge, mid, hi2))

      lo2, hi2 = lax.fori_loop(0, 12, bis_b, (jnp.int32(0), count))
      u = start + lo2

      tv = jnp.full((L,), t, jnp.int32)
      tbits = jnp.where(tv >= 0, tv, INT_MIN - tv)
      obuf[0:L] = lax.bitcast_convert_type(tbits, jnp.float32)
      pltpu.sync_copy(obuf, tf_hbm.at[g])
      obuf[0:L] = jnp.full((L,), u.astype(jnp.float32), jnp.float32)
      pltpu.sync_copy(obuf, u_hbm.at[g])

  k = pl.kernel(
      body,
      out_type=(
          jax.ShapeDtypeStruct((B, L), jnp.float32),
          jax.ShapeDtypeStruct((B, L), jnp.float32),
      ),
      mesh=_mesh(),
      scratch_types=[
          pltpu.VMEM((SG,), jnp.float32),
          pltpu.VMEM((SG,), jnp.int32),
          pltpu.VMEM((L,), jnp.float32),
          pltpu.VMEM((2 * B,), jnp.int32),
      ],
      compiler_params=pltpu.CompilerParams(needs_layout_passes=False),
  )
  return k(skey_pad, sc128)


RB = 1000


def _conv_body(weighted, parts, agg_ref, deg_ref, w_ref, xin_ref, wrel_ref,
               b_ref, wroot_ref, batch_ref, h_ref, xsum_ref):
  i = pl.program_id(0)
  d = deg_ref[0, :, 0:1] + deg_ref[1, :, 0:1]
  if weighted:
    w = w_ref[:, 0:1]
    den = jnp.maximum(d * w, 1.0)
  else:
    w = None
    den = jnp.maximum(d, 1.0)

  def scale(a):
    return ((a * w) if weighted else a) / den

  if parts:
    a = scale(agg_ref[0] + agg_ref[1])
    acc = jnp.dot(a, wrel_ref[...], preferred_element_type=jnp.float32, precision=lax.Precision.HIGHEST)
    acc = acc + jnp.dot(xin_ref[...], wroot_ref[...],
                        preferred_element_type=jnp.float32, precision=lax.Precision.HIGHEST)
  else:
    dh = 128
    alo = scale(agg_ref[0])
    ahi = scale(agg_ref[1])
    acc = jnp.dot(alo, wrel_ref[:dh], preferred_element_type=jnp.float32, precision=lax.Precision.HIGHEST)
    acc = acc + jnp.dot(ahi, wrel_ref[dh:], preferred_element_type=jnp.float32, precision=lax.Precision.HIGHEST)
    acc = acc + jnp.dot(xin_ref[0], wroot_ref[:dh],
                        preferred_element_type=jnp.float32, precision=lax.Precision.HIGHEST)
    acc = acc + jnp.dot(xin_ref[1], wroot_ref[dh:],
                        preferred_element_type=jnp.float32, precision=lax.Precision.HIGHEST)
  acc = acc + b_ref[0:1, :]
  h = jnp.maximum(acc, 0.0)
  if weighted:
    h = h * w
  h_ref[0] = h[:, :128]
  h_ref[1] = h[:, 128:]
  gcol = lax.broadcasted_iota(jnp.int32, (RB, B), 1)
  onehot = (batch_ref[:, 0:1] == gcol).astype(jnp.float32)
  part = lax.dot_general(onehot, h, (((0,), (0,)), ((), ())),
                         preferred_element_type=jnp.float32,
                         precision=lax.Precision.HIGHEST)

  @pl.when(i == 0)
  def _():
    xsum_ref[...] = jnp.zeros_like(xsum_ref)

  xsum_ref[...] += part


def _tc_conv(agg2, degsm, w16, xin, wrel, b, wroot, batch2, weighted, parts):
  grid = (N // RB,)
  din = wrel.shape[0]
  dw = degsm.shape[2]
  xin_spec = (pl.BlockSpec((RB, din), lambda i: (i, 0)) if parts
              else pl.BlockSpec((2, RB, 128), lambda i: (0, i, 0)))
  in_specs = [
      pl.BlockSpec((2, RB, 128), lambda i: (0, i, 0)),
      pl.BlockSpec((2, RB, dw), lambda i: (0, i, 0)),
      pl.BlockSpec((RB, 128), lambda i: (i, 0)),
      xin_spec,
      pl.BlockSpec((din, H), lambda i: (0, 0)),
      pl.BlockSpec((1, H), lambda i: (0, 0)),
      pl.BlockSpec((din, H), lambda i: (0, 0)),
      pl.BlockSpec((RB, 1), lambda i: (i, 0)),
  ]
  out_specs = [
      pl.BlockSpec((2, RB, 128), lambda i: (0, i, 0)),
      pl.BlockSpec((B, H), lambda i: (0, 0)),
  ]
  out_shape = [
      jax.ShapeDtypeStruct((2, N, 128), jnp.float32),
      jax.ShapeDtypeStruct((B, H), jnp.float32),
  ]
  return pl.pallas_call(
      functools.partial(_conv_body, weighted, parts),
      grid=grid, in_specs=in_specs, out_specs=out_specs, out_shape=out_shape,
  )(agg2, degsm, w16, xin, wrel, b, wroot, batch2)


def _score_body(h_ref, p_ref, batch_ref, bfull_ref, score_ref, skey_ref,
                sc_ref):
  i = pl.program_id(0)
  p = p_ref[...]
  nrm = jnp.sqrt(jnp.sum(p * p))
  hp = jnp.dot(h_ref[0], p[:128, :], preferred_element_type=jnp.float32, precision=lax.Precision.HIGHEST)
  hp = hp + jnp.dot(h_ref[1], p[128:, :], preferred_element_type=jnp.float32, precision=lax.Precision.HIGHEST)
  s = jnp.tanh(hp / nrm)
  score_ref[...] = s
  skey_ref[...] = batch_ref[...].astype(jnp.float32) * 4.0 - s

  @pl.when(i == 0)
  def _():
    gcol = lax.broadcasted_iota(jnp.int32, (N, B), 1)
    onehot = (bfull_ref[:, 0:1] == gcol).astype(jnp.float32)
    counts = jnp.sum(onehot, axis=0, keepdims=True)
    rr = lax.broadcasted_iota(jnp.int32, (B, B), 0)
    cc = lax.broadcasted_iota(jnp.int32, (B, B), 1)
    tri = (rr < cc).astype(jnp.float32)
    starts = jnp.dot(counts, tri, preferred_element_type=jnp.float32, precision=lax.Precision.HIGHEST)
    sc_ref[...] = jnp.concatenate([starts, counts], axis=1).astype(jnp.int32)


def _tc_score(h2, p0, batch2):
  grid = (N // RB,)
  return pl.pallas_call(
      _score_body,
      grid=grid,
      in_specs=[
          pl.BlockSpec((2, RB, 128), lambda i: (0, i, 0)),
          pl.BlockSpec((H, 1), lambda i: (0, 0)),
          pl.BlockSpec((RB, 1), lambda i: (i, 0)),
          pl.BlockSpec((N, 1), lambda i: (0, 0)),
      ],
      out_specs=[
          pl.BlockSpec((RB, 1), lambda i: (i, 0)),
          pl.BlockSpec((RB, 1), lambda i: (i, 0)),
          pl.BlockSpec((1, 2 * B), lambda i: (0, 0)),
      ],
      out_shape=[
          jax.ShapeDtypeStruct((N, 1), jnp.float32),
          jax.ShapeDtypeStruct((N, 1), jnp.float32),
          jax.ShapeDtypeStruct((1, 2 * B), jnp.int32),
      ],
  )(h2, p0, batch2, batch2)


def _mask_body(h_ref, score_ref, skey_ref, batch_ref, th_ref, h3m_ref,
               m16_ref):
  i = pl.program_id(0)
  gcol = lax.broadcasted_iota(jnp.int32, (RB, B), 1)
  onehot = (batch_ref[:, 0:1] == gcol).astype(jnp.float32)
  g2 = jnp.dot(onehot, th_ref[...], preferred_element_type=jnp.float32, precision=lax.Precision.HIGHEST)
  tf = g2[:, 0:1]
  uu = g2[:, 1:2]
  sk = skey_ref[...]
  idxrow = (lax.broadcasted_iota(jnp.int32, (RB, 1), 0)
            + i * RB).astype(jnp.float32)
  m = ((sk < tf) | ((sk == tf) & (idxrow < uu))).astype(jnp.float32)
  hm = score_ref[...] * m
  h3m_ref[0] = h_ref[0] * hm
  h3m_ref[1] = h_ref[1] * hm
  m16_ref[...] = jnp.broadcast_to(m, (RB, 128))


def _tc_mask(h2, score, skey, batch2, th3):
  grid = (N // RB,)
  return pl.pallas_call(
      _mask_body,
      grid=grid,
      in_specs=[
          pl.BlockSpec((2, RB, 128), lambda i: (0, i, 0)),
          pl.BlockSpec((RB, 1), lambda i: (i, 0)),
          pl.BlockSpec((RB, 1), lambda i: (i, 0)),
          pl.BlockSpec((RB, 1), lambda i: (i, 0)),
          pl.BlockSpec((B, 2), lambda i: (0, 0)),
      ],
      out_specs=[
          pl.BlockSpec((2, RB, 128), lambda i: (0, i, 0)),
          pl.BlockSpec((RB, 128), lambda i: (i, 0)),
      ],
      out_shape=[
          jax.ShapeDtypeStruct((2, N, 128), jnp.float32),
          jax.ShapeDtypeStruct((N, 128), jnp.float32),
      ],
  )(h2, score, skey, batch2, th3)


def _mlp_body(x0, x1, x2, x3, w1, b1, w2, b2, w3, b3, out_ref):
  z = jnp.dot(x0[...], w1[:H], preferred_element_type=jnp.float32, precision=lax.Precision.HIGHEST)
  z = z + jnp.dot(x1[...], w1[H:2 * H], preferred_element_type=jnp.float32, precision=lax.Precision.HIGHEST)
  z = z + jnp.dot(x2[...], w1[2 * H:3 * H], preferred_element_type=jnp.float32, precision=lax.Precision.HIGHEST)
  z = z + jnp.dot(x3[...], w1[3 * H:], preferred_element_type=jnp.float32, precision=lax.Precision.HIGHEST)
  z = jnp.maximum(z + b1[0:1, :], 0.0)
  z = jnp.maximum(jnp.dot(z, w2[...], preferred_element_type=jnp.float32, precision=lax.Precision.HIGHEST)
                  + b2[0:1, :], 0.0)
  z = jnp.dot(z, w3[...], preferred_element_type=jnp.float32, precision=lax.Precision.HIGHEST) + b3[0:1, :]
  mx = jnp.max(z, axis=1, keepdims=True)
  sh = z - mx
  out_ref[...] = sh - jnp.log(jnp.sum(jnp.exp(sh), axis=1, keepdims=True))


def _tc_mlp(xs, params):
  return pl.pallas_call(
      _mlp_body,
      out_shape=jax.ShapeDtypeStruct((B, C), jnp.float32),
  )(xs[0], xs[1], xs[2], xs[3],
    params['W1'], params['b1'].reshape(1, H),
    params['W2'], params['b2'].reshape(1, H // 2),
    params['W3'], params['b3'].reshape(1, C))


def kernel(x, params, edge_index, batch):
  src = edge_index[0]
  dst = edge_index[1]
  batch2 = batch.reshape(N, 1)

  ones128 = jnp.ones((CH, 128), jnp.float32)
  zrs128 = jnp.zeros((128, 128), jnp.float32)
  deg = _sc_degree(ones128, zrs128, dst).reshape(2, NPAD, 128)

  agg1 = _sc_segsum_feat(x, src, dst, False).reshape(2, NPAD, 128)
  h1, xs0 = _tc_conv(agg1, deg, x, x, params['W_rel1'],
                     params['b_rel1'].reshape(1, H), params['W_root1'],
                     batch2, False, True)

  h1_flat = h1.reshape(2 * N, 128)
  agg2 = _sc_segsum_feat(h1_flat, src, dst, True).reshape(2, NPAD, 128)
  h2, xs1 = _tc_conv(agg2, deg, agg2[0, :N], h1, params['W_rel2'],
                     params['b_rel2'].reshape(1, H), params['W_root2'],
                     batch2, False, False)

  score, skey, sc128 = _tc_score(h2, params['p0'].reshape(H, 1), batch2)
  skey_pad = jnp.concatenate(
      [skey.reshape(N), jnp.full((SKEY_PAD,), 1e30, jnp.float32)])
  tf16, u16 = _sc_topk(skey_pad, sc128.reshape(2 * B))
  th2 = jnp.concatenate([tf16[:, 0:1], u16[:, 0:1]], axis=1)

  h3m, m16 = _tc_mask(h2, score, skey, batch2, th2)

  sm = _sc_segsum_feat(m16, src, dst, False).reshape(2, NPAD, 128)

  h3m_flat = h3m.reshape(2 * N, 128)
  agg3 = _sc_segsum_feat(h3m_flat, src, dst, True).reshape(2, NPAD, 128)
  h4m, xs2 = _tc_conv(agg3, sm, m16, h3m, params['W_rel3'],
                      params['b_rel3'].reshape(1, H), params['W_root3'],
                      batch2, True, False)

  h4m_flat = h4m.reshape(2 * N, 128)
  agg4 = _sc_segsum_feat(h4m_flat, src, dst, True).reshape(2, NPAD, 128)
  _, xs3 = _tc_conv(agg4, sm, m16, h4m, params['W_rel4'],
                    params['b_rel4'].reshape(1, H), params['W_root4'],
                    batch2, True, False)

  return _tc_mlp([xs0, xs1, xs2, xs3], params)


# speedup vs baseline: 10.3578x; 1.1681x over previous
import functools

import jax
import jax.numpy as jnp
from jax import lax
from jax.experimental import pallas as pl
from jax.experimental.pallas import tpu as pltpu
from jax.experimental.pallas import tpu_sc as plsc

N, E, F, H, B, C = 10000, 320000, 128, 256, 64, 10

NC, NS, L = 2, 16, 16
NPAD = 10240
CH = 80
SKEY_PAD = 1024
SG = 544
INT_MIN = -2147483648

_mesh = functools.partial(
    plsc.VectorSubcoreMesh, core_axis_name="c", subcore_axis_name="s",
    num_cores=NC, num_subcores=NS)


def _f32key_to_i32(b):
  return jnp.where(b >= 0, b, INT_MIN - b)


def _sc_degree(ones128, zrs128, dst):
  dh = 128
  nch = E // (NC * NS * CH)

  def body(ones_hbm, zrs_hbm, dst_hbm, out_hbm, ones_v, dstb, d0, acc, wb):
    c = lax.axis_index("c")
    s = lax.axis_index("s")
    rows0 = s * (NPAD // NS)

    pltpu.sync_copy(ones_hbm, ones_v)
    pltpu.sync_copy(zrs_hbm, wb)
    for t in range(5):
      pltpu.sync_copy(wb, acc.at[pl.ds(rows0 + 128 * t, 128)])
    base0 = (s * NC + c) * nch * CH
    pltpu.sync_copy(dst_hbm.at[pl.ds(base0, nch * CH)], dstb)
    plsc.subcore_barrier()

    def chunk(j, _):
      for kk in range(CH // L):
        sl = pl.multiple_of(j * CH + kk * L, 8)
        d0[pl.ds(kk * L, L)] = dstb[pl.ds(sl, L)]
      pltpu.sync_copy(ones_v, acc.at[d0], add=True)
      return 0
    lax.fori_loop(0, nch, chunk, 0)
    plsc.subcore_barrier()

    for t in range(5):
      pltpu.sync_copy(acc.at[pl.ds(rows0 + 128 * t, 128)], wb)
      pltpu.sync_copy(wb, out_hbm.at[pl.ds(c * NPAD + rows0 + 128 * t, 128)])

  k = pl.kernel(
      body,
      out_type=jax.ShapeDtypeStruct((NC * NPAD, dh), jnp.float32),
      mesh=_mesh(),
      scratch_types=[
          pltpu.VMEM((CH, dh), jnp.float32),
          pltpu.VMEM((E // (NC * NS),), jnp.int32),
          pltpu.VMEM((CH,), jnp.int32),
          pltpu.VMEM_SHARED((NPAD, dh), jnp.float32),
          pltpu.VMEM((128, dh), jnp.float32),
      ],
  )
  return k(ones128, zrs128, dst)


def _sc_segsum_feat(tab, src, dst, feat_split):
  dh = 128
  pw = E // NS if feat_split else E // (NC * NS)
  nch = pw // CH
  G = 25
  nblk = nch // G
  assert nblk * G == nch

  def body(tab_hbm, src_hbm, dst_hbm, out_hbm,
           srcb, dstb, g0, g1, g2, g3, d0, d1, d2, d3,
           rows0, rows1, rows2, rows3, acc,
           sem0, sem1, sem2, sem3, ssem0, ssem1, ssem2, ssem3):
    c = lax.axis_index("c")
    s = lax.axis_index("s")
    rows_base = s * (NPAD // NS)

    def zrow(i, _):
      def zlane(k, __):
        rows0[i, pl.ds(k * L, L)] = jnp.zeros((L,), jnp.float32)
        return 0
      lax.fori_loop(0, dh // L, zlane, 0)
      return 0
    lax.fori_loop(0, CH, zrow, 0)
    for t in range(8):
      pltpu.sync_copy(rows0, acc.at[pl.ds(rows_base + CH * t, CH)])

    if feat_split:
      base0 = s * pw
      coff = c * N
    else:
      base0 = (s * NC + c) * pw
      coff = None
    plsc.subcore_barrier()

    offv = (jnp.full((L,), coff, jnp.int32) if coff is not None
            else jnp.zeros((L,), jnp.int32))

    bufs = ((g0, d0, rows0, sem0, ssem0), (g1, d1, rows1, sem1, ssem1),
            (g2, d2, rows2, sem2, ssem2), (g3, d3, rows3, sem3, ssem3))
    for (gb, db, rb, sm, ssm) in bufs:
      def zr(i, _, rb=rb):
        for k2 in range(dh // L):
          rb[i, pl.ds(k2 * L, L)] = jnp.zeros((L,), jnp.float32)
        return 0
      lax.fori_loop(0, CH, zr, 0)
      for k2 in range(CH // L):
        db[pl.ds(k2 * L, L)] = jnp.zeros((L,), jnp.int32)
      pltpu.async_copy(rb, acc.at[db], ssm, add=True)

    def load_idx(jj, gb, db):
      for kk in range(CH // L):
        sl = pl.multiple_of(jj * CH + kk * L, 8)
        gb[pl.ds(kk * L, L)] = srcb[pl.ds(sl, L)] + offv
        db[pl.ds(kk * L, L)] = dstb[pl.ds(sl, L)]

    def blk(bi, _):
      boff = base0 + bi * (G * CH)
      pltpu.sync_copy(src_hbm.at[pl.ds(boff, G * CH)], srcb)
      pltpu.sync_copy(dst_hbm.at[pl.ds(boff, G * CH)], dstb)

      def quad(q, _):
        j0 = 4 * q
        cps = []
        for b, (gb, db, rb, sm, ssm) in enumerate(bufs):
          pltpu.make_async_copy(rb, acc.at[db], ssm).wait()
          load_idx(j0 + b, gb, db)
          cps.append(pltpu.async_copy(tab_hbm.at[gb], rb, sm))
        for b, (gb, db, rb, sm, ssm) in enumerate(bufs):
          cps[b].wait()
          pltpu.async_copy(rb, acc.at[db], ssm, add=True)
        return 0
      lax.fori_loop(0, G // 4, quad, 0)
      gb, db, rb, sm, ssm = bufs[0]
      pltpu.make_async_copy(rb, acc.at[db], ssm).wait()
      load_idx(G - 1, gb, db)
      pltpu.async_copy(tab_hbm.at[gb], rb, sm).wait()
      pltpu.async_copy(rb, acc.at[db], ssm, add=True)
      return 0
    lax.fori_loop(0, nblk, blk, 0)
    for (gb, db, rb, sm, ssm) in bufs:
      pltpu.make_async_copy(rb, acc.at[db], ssm).wait()
    plsc.subcore_barrier()

    for t in range(8):
      pltpu.sync_copy(acc.at[pl.ds(rows_base + CH * t, CH)], rows0)
      pltpu.sync_copy(
          rows0, out_hbm.at[pl.ds(c * NPAD + rows_base + CH * t, CH)])

  k = pl.kernel(
      body,
      out_type=jax.ShapeDtypeStruct((NC * NPAD, dh), jnp.float32),
      mesh=_mesh(),
      scratch_types=(
          [pltpu.VMEM((G * CH,), jnp.int32)] * 2
          + [pltpu.VMEM((CH,), jnp.int32)] * 8
          + [pltpu.VMEM((CH, dh), jnp.float32)] * 4
          + [pltpu.VMEM_SHARED((NPAD, dh), jnp.float32)]
          + [pltpu.SemaphoreType.DMA] * 8
      ),
  )
  return k(tab, src, dst)


def _sc_topk(skey_pad, sc128):
  nv = SG // L

  def body(skey_hbm, sc_hbm, tf_hbm, u_hbm,
           kbuf, keyi, obuf, scm):
    c = lax.axis_index("c")
    s = lax.axis_index("s")
    wid = s * NC + c
    pltpu.sync_copy(sc_hbm, scm)

    iota = lax.iota(jnp.int32, L)

    def scread(idx):
      base = pl.multiple_of((idx // L) * L, 8)
      v = scm[pl.ds(base, L)]
      return jnp.sum(jnp.where(iota == idx - base, v, 0))

    for g_loc in range(2):
      g = wid * 2 + g_loc
      start = scread(g)
      count = scread(B + g)
      a = pl.multiple_of(lax.bitwise_and(start, jnp.int32(-8)), 8)
      off = start - a
      pltpu.sync_copy(skey_hbm.at[pl.ds(a, SG)], kbuf)

      def conv(j, _):
        sl = pl.ds(j * L, L)
        bits = lax.bitcast_convert_type(kbuf[sl], jnp.int32)
        keyi[sl] = _f32key_to_i32(bits)
        return 0
      lax.fori_loop(0, nv, conv, 0)

      offv = jnp.full((L,), off, jnp.int32)
      cntv = jnp.full((L,), count, jnp.int32)
      kq = 4 * count + 4
      kper = lax.shift_right_logical(kq * 52429, 18)

      def count_pred(pred):
        def inner(j, acc):
          sl = pl.ds(j * L, L)
          kv = keyi[sl]
          pos = jnp.full((L,), j * L, jnp.int32) + iota - offv
          valid = (pos >= 0) & (pos < cntv)
          return acc + (pred(kv, pos) & valid).astype(jnp.int32)
        lanes = lax.fori_loop(0, nv, inner, jnp.zeros((L,), jnp.int32))
        return jnp.sum(lanes)

      def bis_a(it, lh):
        lo, hi = lh
        mid = (lax.shift_right_arithmetic(lo, 1)
               + lax.shift_right_arithmetic(hi, 1)
               + (lo & hi & 1))
        ge = count_pred(lambda kv, pos: kv <= mid) >= kper
        return (jnp.where(ge, lo, mid + 1), jnp.where(ge, mid, hi))

      lo, hi = lax.fori_loop(
          0, 32, bis_a,
          (jnp.int32(INT_MIN), jnp.int32(2147483647)))
      t = lo

      strict = count_pred(lambda kv, pos: kv < t)
      r = kper - strict

      def bis_b(it, lh):
        lo2, hi2 = lh
        mid = lax.shift_right_arithmetic(lo2 + hi2, 1)
        ge = count_pred(lambda kv, pos: (kv == t) & (pos < mid)) >= r
        return (jnp.where(ge, lo2, mid + 1), jnp.where(ge, mid, hi2))

      lo2, hi2 = lax.fori_loop(0, 12, bis_b, (jnp.int32(0), count))
      u = start + lo2

      tv = jnp.full((L,), t, jnp.int32)
      tbits = jnp.where(tv >= 0, tv, INT_MIN - tv)
      obuf[0:L] = lax.bitcast_convert_type(tbits, jnp.float32)
      pltpu.sync_copy(obuf, tf_hbm.at[g])
      obuf[0:L] = jnp.full((L,), u.astype(jnp.float32), jnp.float32)
      pltpu.sync_copy(obuf, u_hbm.at[g])

  k = pl.kernel(
      body,
      out_type=(
          jax.ShapeDtypeStruct((B, L), jnp.float32),
          jax.ShapeDtypeStruct((B, L), jnp.float32),
      ),
      mesh=_mesh(),
      scratch_types=[
          pltpu.VMEM((SG,), jnp.float32),
          pltpu.VMEM((SG,), jnp.int32),
          pltpu.VMEM((L,), jnp.float32),
          pltpu.VMEM((2 * B,), jnp.int32),
      ],
      compiler_params=pltpu.CompilerParams(needs_layout_passes=False),
  )
  return k(skey_pad, sc128)


RB = 1000


def _conv_body(weighted, parts, agg_ref, deg_ref, w_ref, xin_ref, wrel_ref,
               b_ref, wroot_ref, batch_ref, h_ref, xsum_ref):
  i = pl.program_id(0)
  d = deg_ref[0, :, 0:1] + deg_ref[1, :, 0:1]
  if weighted:
    w = w_ref[:, 0:1]
    den = jnp.maximum(d * w, 1.0)
  else:
    w = None
    den = jnp.maximum(d, 1.0)

  def scale(a):
    return ((a * w) if weighted else a) / den

  if parts:
    a = scale(agg_ref[0] + agg_ref[1])
    acc = jnp.dot(a, wrel_ref[...], preferred_element_type=jnp.float32, precision=lax.Precision.HIGHEST)
    acc = acc + jnp.dot(xin_ref[...], wroot_ref[...],
                        preferred_element_type=jnp.float32, precision=lax.Precision.HIGHEST)
  else:
    dh = 128
    alo = scale(agg_ref[0])
    ahi = scale(agg_ref[1])
    acc = jnp.dot(alo, wrel_ref[:dh], preferred_element_type=jnp.float32, precision=lax.Precision.HIGHEST)
    acc = acc + jnp.dot(ahi, wrel_ref[dh:], preferred_element_type=jnp.float32, precision=lax.Precision.HIGHEST)
    acc = acc + jnp.dot(xin_ref[0], wroot_ref[:dh],
                        preferred_element_type=jnp.float32, precision=lax.Precision.HIGHEST)
    acc = acc + jnp.dot(xin_ref[1], wroot_ref[dh:],
                        preferred_element_type=jnp.float32, precision=lax.Precision.HIGHEST)
  acc = acc + b_ref[0:1, :]
  h = jnp.maximum(acc, 0.0)
  if weighted:
    h = h * w
  h_ref[0] = h[:, :128]
  h_ref[1] = h[:, 128:]
  gcol = lax.broadcasted_iota(jnp.int32, (RB, B), 1)
  onehot = (batch_ref[:, 0:1] == gcol).astype(jnp.float32)
  part = lax.dot_general(onehot, h, (((0,), (0,)), ((), ())),
                         preferred_element_type=jnp.float32,
                         precision=lax.Precision.HIGHEST)

  @pl.when(i == 0)
  def _():
    xsum_ref[...] = jnp.zeros_like(xsum_ref)

  xsum_ref[...] += part


def _tc_conv(agg2, degsm, w16, xin, wrel, b, wroot, batch2, weighted, parts):
  grid = (N // RB,)
  din = wrel.shape[0]
  dw = degsm.shape[2]
  xin_spec = (pl.BlockSpec((RB, din), lambda i: (i, 0)) if parts
              else pl.BlockSpec((2, RB, 128), lambda i: (0, i, 0)))
  in_specs = [
      pl.BlockSpec((2, RB, 128), lambda i: (0, i, 0)),
      pl.BlockSpec((2, RB, dw), lambda i: (0, i, 0)),
      pl.BlockSpec((RB, 128), lambda i: (i, 0)),
      xin_spec,
      pl.BlockSpec((din, H), lambda i: (0, 0)),
      pl.BlockSpec((1, H), lambda i: (0, 0)),
      pl.BlockSpec((din, H), lambda i: (0, 0)),
      pl.BlockSpec((RB, 1), lambda i: (i, 0)),
  ]
  out_specs = [
      pl.BlockSpec((2, RB, 128), lambda i: (0, i, 0)),
      pl.BlockSpec((B, H), lambda i: (0, 0)),
  ]
  out_shape = [
      jax.ShapeDtypeStruct((2, N, 128), jnp.float32),
      jax.ShapeDtypeStruct((B, H), jnp.float32),
  ]
  return pl.pallas_call(
      functools.partial(_conv_body, weighted, parts),
      grid=grid, in_specs=in_specs, out_specs=out_specs, out_shape=out_shape,
  )(agg2, degsm, w16, xin, wrel, b, wroot, batch2)


def _score_body(h_ref, p_ref, batch_ref, bfull_ref, score_ref, skey_ref,
                sc_ref):
  i = pl.program_id(0)
  p = p_ref[...]
  nrm = jnp.sqrt(jnp.sum(p * p))
  hp = jnp.dot(h_ref[0], p[:128, :], preferred_element_type=jnp.float32, precision=lax.Precision.HIGHEST)
  hp = hp + jnp.dot(h_ref[1], p[128:, :], preferred_element_type=jnp.float32, precision=lax.Precision.HIGHEST)
  s = jnp.tanh(hp / nrm)
  score_ref[...] = s
  skey_ref[...] = batch_ref[...].astype(jnp.float32) * 4.0 - s

  @pl.when(i == 0)
  def _():
    gcol = lax.broadcasted_iota(jnp.int32, (N, B), 1)
    onehot = (bfull_ref[:, 0:1] == gcol).astype(jnp.float32)
    counts = jnp.sum(onehot, axis=0, keepdims=True)
    rr = lax.broadcasted_iota(jnp.int32, (B, B), 0)
    cc = lax.broadcasted_iota(jnp.int32, (B, B), 1)
    tri = (rr < cc).astype(jnp.float32)
    starts = jnp.dot(counts, tri, preferred_element_type=jnp.float32, precision=lax.Precision.HIGHEST)
    sc_ref[...] = jnp.concatenate([starts, counts], axis=1).astype(jnp.int32)


def _tc_score(h2, p0, batch2):
  grid = (N // RB,)
  return pl.pallas_call(
      _score_body,
      grid=grid,
      in_specs=[
          pl.BlockSpec((2, RB, 128), lambda i: (0, i, 0)),
          pl.BlockSpec((H, 1), lambda i: (0, 0)),
          pl.BlockSpec((RB, 1), lambda i: (i, 0)),
          pl.BlockSpec((N, 1), lambda i: (0, 0)),
      ],
      out_specs=[
          pl.BlockSpec((RB, 1), lambda i: (i, 0)),
          pl.BlockSpec((RB, 1), lambda i: (i, 0)),
          pl.BlockSpec((1, 2 * B), lambda i: (0, 0)),
      ],
      out_shape=[
          jax.ShapeDtypeStruct((N, 1), jnp.float32),
          jax.ShapeDtypeStruct((N, 1), jnp.float32),
          jax.ShapeDtypeStruct((1, 2 * B), jnp.int32),
      ],
  )(h2, p0, batch2, batch2)


def _mask_body(h_ref, score_ref, skey_ref, batch_ref, th_ref, h3m_ref,
               m16_ref):
  i = pl.program_id(0)
  gcol = lax.broadcasted_iota(jnp.int32, (RB, B), 1)
  onehot = (batch_ref[:, 0:1] == gcol).astype(jnp.float32)
  g2 = jnp.dot(onehot, th_ref[...], preferred_element_type=jnp.float32, precision=lax.Precision.HIGHEST)
  tf = g2[:, 0:1]
  uu = g2[:, 1:2]
  sk = skey_ref[...]
  idxrow = (lax.broadcasted_iota(jnp.int32, (RB, 1), 0)
            + i * RB).astype(jnp.float32)
  m = ((sk < tf) | ((sk == tf) & (idxrow < uu))).astype(jnp.float32)
  hm = score_ref[...] * m
  h3m_ref[0] = h_ref[0] * hm
  h3m_ref[1] = h_ref[1] * hm
  m16_ref[...] = jnp.broadcast_to(m, (RB, 128))


def _tc_mask(h2, score, skey, batch2, th3):
  grid = (N // RB,)
  return pl.pallas_call(
      _mask_body,
      grid=grid,
      in_specs=[
          pl.BlockSpec((2, RB, 128), lambda i: (0, i, 0)),
          pl.BlockSpec((RB, 1), lambda i: (i, 0)),
          pl.BlockSpec((RB, 1), lambda i: (i, 0)),
          pl.BlockSpec((RB, 1), lambda i: (i, 0)),
          pl.BlockSpec((B, 2), lambda i: (0, 0)),
      ],
      out_specs=[
          pl.BlockSpec((2, RB, 128), lambda i: (0, i, 0)),
          pl.BlockSpec((RB, 128), lambda i: (i, 0)),
      ],
      out_shape=[
          jax.ShapeDtypeStruct((2, N, 128), jnp.float32),
          jax.ShapeDtypeStruct((N, 128), jnp.float32),
      ],
  )(h2, score, skey, batch2, th3)


def _mlp_body(x0, x1, x2, x3, w1, b1, w2, b2, w3, b3, out_ref):
  z = jnp.dot(x0[...], w1[:H], preferred_element_type=jnp.float32, precision=lax.Precision.HIGHEST)
  z = z + jnp.dot(x1[...], w1[H:2 * H], preferred_element_type=jnp.float32, precision=lax.Precision.HIGHEST)
  z = z + jnp.dot(x2[...], w1[2 * H:3 * H], preferred_element_type=jnp.float32, precision=lax.Precision.HIGHEST)
  z = z + jnp.dot(x3[...], w1[3 * H:], preferred_element_type=jnp.float32, precision=lax.Precision.HIGHEST)
  z = jnp.maximum(z + b1[0:1, :], 0.0)
  z = jnp.maximum(jnp.dot(z, w2[...], preferred_element_type=jnp.float32, precision=lax.Precision.HIGHEST)
                  + b2[0:1, :], 0.0)
  z = jnp.dot(z, w3[...], preferred_element_type=jnp.float32, precision=lax.Precision.HIGHEST) + b3[0:1, :]
  mx = jnp.max(z, axis=1, keepdims=True)
  sh = z - mx
  out_ref[...] = sh - jnp.log(jnp.sum(jnp.exp(sh), axis=1, keepdims=True))


def _tc_mlp(xs, params):
  return pl.pallas_call(
      _mlp_body,
      out_shape=jax.ShapeDtypeStruct((B, C), jnp.float32),
  )(xs[0], xs[1], xs[2], xs[3],
    params['W1'], params['b1'].reshape(1, H),
    params['W2'], params['b2'].reshape(1, H // 2),
    params['W3'], params['b3'].reshape(1, C))


def kernel(x, params, edge_index, batch):
  src = edge_index[0]
  dst = edge_index[1]
  batch2 = batch.reshape(N, 1)

  ones128 = jnp.ones((CH, 128), jnp.float32)
  zrs128 = jnp.zeros((128, 128), jnp.float32)
  deg = _sc_degree(ones128, zrs128, dst).reshape(2, NPAD, 128)

  agg1 = _sc_segsum_feat(x, src, dst, False).reshape(2, NPAD, 128)
  h1, xs0 = _tc_conv(agg1, deg, x, x, params['W_rel1'],
                     params['b_rel1'].reshape(1, H), params['W_root1'],
                     batch2, False, True)

  h1_flat = h1.reshape(2 * N, 128)
  agg2 = _sc_segsum_feat(h1_flat, src, dst, True).reshape(2, NPAD, 128)
  h2, xs1 = _tc_conv(agg2, deg, agg2[0, :N], h1, params['W_rel2'],
                     params['b_rel2'].reshape(1, H), params['W_root2'],
                     batch2, False, False)

  score, skey, sc128 = _tc_score(h2, params['p0'].reshape(H, 1), batch2)
  skey_pad = jnp.concatenate(
      [skey.reshape(N), jnp.full((SKEY_PAD,), 1e30, jnp.float32)])
  tf16, u16 = _sc_topk(skey_pad, sc128.reshape(2 * B))
  th2 = jnp.concatenate([tf16[:, 0:1], u16[:, 0:1]], axis=1)

  h3m, m16 = _tc_mask(h2, score, skey, batch2, th2)

  sm = _sc_segsum_feat(m16, src, dst, False).reshape(2, NPAD, 128)

  h3m_flat = h3m.reshape(2 * N, 128)
  agg3 = _sc_segsum_feat(h3m_flat, src, dst, True).reshape(2, NPAD, 128)
  h4m, xs2 = _tc_conv(agg3, sm, m16, h3m, params['W_rel3'],
                      params['b_rel3'].reshape(1, H), params['W_root3'],
                      batch2, True, False)

  h4m_flat = h4m.reshape(2 * N, 128)
  agg4 = _sc_segsum_feat(h4m_flat, src, dst, True).reshape(2, NPAD, 128)
  _, xs3 = _tc_conv(agg4, sm, m16, h4m, params['W_rel4'],
                    params['b_rel4'].reshape(1, H), params['W_root4'],
                    batch2, True, False)

  return _tc_mlp([xs0, xs1, xs2, xs3], params)
